# baseline (speedup 1.0000x reference)
import functools

import jax
import jax.numpy as jnp
from jax import lax
from jax.experimental import pallas as pl
from jax.experimental.pallas import tpu as pltpu
from jax.experimental.pallas import tpu_sc as plsc

N = 10000
E = 320000
D = 128
L = 16
NC = 2
NS = 16
N_PAD = 10240
ROWS_PER_TILE = N_PAD // NS
E_PAD = 327680
E_TILE = E_PAD // (NC * NS)
B_E = 128
N_EBATCH = E_TILE // B_E
NBUF = 2
CORE0_BATCHES = 48
CORE1_BATCHES = 2 * N_EBATCH - CORE0_BATCHES
CHUNK_R = 16
E_CNT_TILE = E_PAD // NS
CNT_ROWS = E_CNT_TILE // B_E
CNT_CHUNK_ROWS = 16
CROWS = N_PAD // B_E
CROWS_PER_TILE = CROWS // NS
ZROWS = 16
ROW_BLK = 1000

_mesh = functools.partial(
    plsc.VectorSubcoreMesh,
    core_axis_name="c", subcore_axis_name="s", num_cores=NC, num_subcores=NS)


def _zero_rows(ref, nrows):
  zero16 = jnp.zeros((L,), jnp.float32)
  def row(i, _):
    def col(q, _):
      ref[i, pl.ds(q * L, L)] = zero16
      return 0
    return lax.fori_loop(0, D // L, col, 0)
  lax.fori_loop(0, nrows, row, 0)


def _edge_scatter_loop(y_hbm, src2_hbm, dst2_hbm, acc_sh, src2_v, dst2_v,
                       rows_bufs, sems, c, s):
  nb = jnp.where(c == 0, CORE0_BATCHES, CORE1_BATCHES)
  row0 = c * (NS * CORE0_BATCHES) + s * nb
  def chunk_loop(ch, _):
    cr0 = row0 + ch * CHUNK_R
    pltpu.sync_copy(src2_hbm.at[pl.ds(cr0, CHUNK_R)], src2_v)
    pltpu.sync_copy(dst2_hbm.at[pl.ds(cr0, CHUNK_R)], dst2_v)
    for b in range(NBUF):
      pltpu.async_copy(y_hbm.at[src2_v.at[b]], rows_bufs[b], sems[b])
    def step(g0, _):
      for b in range(NBUF):
        g = g0 * NBUF + b
        pltpu.make_async_copy(
            y_hbm.at[src2_v.at[g]], rows_bufs[b], sems[b]).wait()
        pltpu.sync_copy(rows_bufs[b], acc_sh.at[dst2_v.at[g]], add=True)
        pltpu.async_copy(y_hbm.at[src2_v.at[g + NBUF]], rows_bufs[b], sems[b])
      return 0
    lax.fori_loop(0, CHUNK_R // NBUF - 1, step, 0)
    for b in range(NBUF):
      g = CHUNK_R - NBUF + b
      pltpu.make_async_copy(
          y_hbm.at[src2_v.at[g]], rows_bufs[b], sems[b]).wait()
      pltpu.sync_copy(rows_bufs[b], acc_sh.at[dst2_v.at[g]], add=True)
    return 0
  lax.fori_loop(0, nb // CHUNK_R, chunk_loop, 0)


def _scale_and_emit(acc_sh, inv_v, stage_v, p_hbm, c, r0):
  def chunk(j, _):
    pltpu.sync_copy(acc_sh.at[pl.ds(r0 + j * ZROWS, ZROWS)], stage_v)
    def row(rr, _):
      idx16 = jnp.full((L,), j * ZROWS + rr, jnp.int32)
      g = plsc.load_gather(inv_v, [idx16])
      def col(q, _):
        stage_v[rr, pl.ds(q * L, L)] = stage_v[rr, pl.ds(q * L, L)] * g
        return 0
      return lax.fori_loop(0, D // L, col, 0)
    lax.fori_loop(0, ZROWS, row, 0)
    pltpu.sync_copy(stage_v, p_hbm.at[c, pl.ds(r0 + j * ZROWS, ZROWS)])
    return 0
  lax.fori_loop(0, ROWS_PER_TILE // ZROWS, chunk, 0)


def _zero_acc(acc_sh, stage_v, r0):
  def zacc(j, _):
    pltpu.sync_copy(stage_v, acc_sh.at[pl.ds(r0 + j * ZROWS, ZROWS)])
    return 0
  lax.fori_loop(0, ROWS_PER_TILE // ZROWS, zacc, 0)


def _sc_agg1_body(y_hbm, src2_hbm, dst2_hbm, p_hbm, inv_hbm,
                  acc_sh, cnt_sh, src2_v, dst2_v, cnt5_v, inv_v,
                  sem0, sem1):
  c = lax.axis_index("c")
  s = lax.axis_index("s")
  r0 = s * ROWS_PER_TILE
  zero16 = jnp.zeros((L,), jnp.float32)
  ones16 = jnp.full((L,), 1.0, jnp.float32)

  def phase_a(stage_v, cntloc_v, dstbuf_v, iota_v):
    _zero_rows(stage_v, ZROWS)
    def zcnt(i, _):
      def zcntc(q, _):
        cntloc_v[i, pl.ds(q * L, L)] = zero16
        return 0
      return lax.fori_loop(0, B_E // L, zcntc, 0)
    lax.fori_loop(0, CROWS, zcnt, 0)
    _zero_acc(acc_sh, stage_v, r0)
    pltpu.sync_copy(stage_v.at[pl.ds(0, CROWS_PER_TILE)],
                    cnt_sh.at[pl.ds(s * CROWS_PER_TILE, CROWS_PER_TILE)])

    t0r = s * CNT_ROWS
    def cnt_chunk(jc, _):
      pltpu.sync_copy(
          dst2_hbm.at[pl.ds(t0r + jc * CNT_CHUNK_ROWS, CNT_CHUNK_ROWS)],
          dstbuf_v)
      def cnt_row(rr, _):
        def cnt_col(q, _):
          d16 = dstbuf_v[rr, pl.ds(q * L, L)]
          row16 = lax.shift_right_logical(d16, 7)
          col16 = lax.bitwise_and(d16, B_E - 1)
          plsc.addupdate_scatter(cntloc_v, [row16, col16], ones16)
          return 0
        return lax.fori_loop(0, B_E // L, cnt_col, 0)
      return lax.fori_loop(0, CNT_CHUNK_ROWS, cnt_row, 0)
    lax.fori_loop(0, CNT_ROWS // CNT_CHUNK_ROWS, cnt_chunk, 0)

    def mkiota(k, _):
      iota_v[pl.ds(k * L, L)] = lax.iota(jnp.int32, L) + k * L
      return 0
    lax.fori_loop(0, CROWS // L, mkiota, 0)

    plsc.subcore_barrier()
    pltpu.sync_copy(cntloc_v, cnt_sh.at[iota_v], add=True)

  pl.run_scoped(
      phase_a,
      pltpu.VMEM((ZROWS, D), jnp.float32),
      pltpu.VMEM((CROWS, B_E), jnp.float32),
      pltpu.VMEM((CNT_CHUNK_ROWS, B_E), jnp.int32),
      pltpu.VMEM((CROWS,), jnp.int32),
  )

  def phase_b(rows0, rows1):
    _edge_scatter_loop(y_hbm, src2_hbm, dst2_hbm, acc_sh, src2_v, dst2_v,
                       (rows0, rows1), (sem0, sem1), c, s)

  pl.run_scoped(phase_b,
                pltpu.VMEM((B_E, D), jnp.float32),
                pltpu.VMEM((B_E, D), jnp.float32))

  plsc.subcore_barrier()

  pltpu.sync_copy(cnt_sh.at[pl.ds(s * CROWS_PER_TILE, CROWS_PER_TILE)],
                  cnt5_v)
  def invrow(j, _):
    def invcol(q, _):
      v = cnt5_v[j, pl.ds(q * L, L)]
      inv_v[pl.ds((j * (B_E // L) + q) * L, L)] = ones16 / jnp.maximum(v, ones16)
      return 0
    return lax.fori_loop(0, B_E // L, invcol, 0)
  lax.fori_loop(0, CROWS_PER_TILE, invrow, 0)

  @pl.when(c == 0)
  def _():
    pltpu.sync_copy(inv_v, inv_hbm.at[pl.ds(r0, ROWS_PER_TILE)])

  def phase_c(stage_v):
    _scale_and_emit(acc_sh, inv_v, stage_v, p_hbm, c, r0)
  pl.run_scoped(phase_c, pltpu.VMEM((ZROWS, D), jnp.float32))


def _sc_agg2_body(y_hbm, src2_hbm, dst2_hbm, inv_hbm, p_hbm,
                  acc_sh, src2_v, dst2_v, stage_v, inv_v,
                  rows0, rows1, sem0, sem1):
  c = lax.axis_index("c")
  s = lax.axis_index("s")
  r0 = s * ROWS_PER_TILE

  _zero_rows(stage_v, ZROWS)
  _zero_acc(acc_sh, stage_v, r0)
  pltpu.sync_copy(inv_hbm.at[pl.ds(r0, ROWS_PER_TILE)], inv_v)

  plsc.subcore_barrier()

  _edge_scatter_loop(y_hbm, src2_hbm, dst2_hbm, acc_sh, src2_v, dst2_v,
                     (rows0, rows1), (sem0, sem1), c, s)

  plsc.subcore_barrier()

  _scale_and_emit(acc_sh, inv_v, stage_v, p_hbm, c, r0)


_ROWBUFS = [pltpu.VMEM((B_E, D), jnp.float32)] * NBUF
_SEMS = [pltpu.SemaphoreType.DMA] * NBUF


@functools.lru_cache(maxsize=None)
def _sc_agg1():
  return pl.kernel(
    _sc_agg1_body,
    out_type=(jax.ShapeDtypeStruct((NC, N_PAD, D), jnp.float32),
              jax.ShapeDtypeStruct((N_PAD,), jnp.float32)),
    mesh=_mesh(),
    compiler_params=pltpu.CompilerParams(needs_layout_passes=False),
    scratch_types=[
        pltpu.VMEM_SHARED((N_PAD, D), jnp.float32),
        pltpu.VMEM_SHARED((CROWS, B_E), jnp.float32),
        pltpu.VMEM((CHUNK_R, B_E), jnp.int32),
        pltpu.VMEM((CHUNK_R, B_E), jnp.int32),
        pltpu.VMEM((CROWS_PER_TILE, B_E), jnp.float32),
        pltpu.VMEM((ROWS_PER_TILE,), jnp.float32),
        *_SEMS,
    ],
  )


@functools.lru_cache(maxsize=None)
def _sc_agg2():
  return pl.kernel(
    _sc_agg2_body,
    out_type=jax.ShapeDtypeStruct((NC, N_PAD, D), jnp.float32),
    mesh=_mesh(),
    compiler_params=pltpu.CompilerParams(needs_layout_passes=False),
    scratch_types=[
        pltpu.VMEM_SHARED((N_PAD, D), jnp.float32),
        pltpu.VMEM((CHUNK_R, B_E), jnp.int32),
        pltpu.VMEM((CHUNK_R, B_E), jnp.int32),
        pltpu.VMEM((ZROWS, D), jnp.float32),
        pltpu.VMEM((ROWS_PER_TILE,), jnp.float32),
        *_ROWBUFS,
        *_SEMS,
    ],
  )


def _tc_a_body(x_ref, wl_ref, wr_ref, y_ref, r_ref):
  xb = x_ref[...]
  y_ref[...] = jnp.dot(xb, wl_ref[...], preferred_element_type=jnp.float32)
  r_ref[...] = jnp.dot(xb, wr_ref[...], preferred_element_type=jnp.float32)


def _tc_b_body(p_ref, r_ref, b_ref, wl_ref, wr_ref, y2_ref, r2_ref):
  sb = p_ref[0] + p_ref[1] + r_ref[...] + b_ref[...][None, :]
  h = jnp.where(sb > 0, sb, jnp.exp(sb) - 1.0)
  y2_ref[...] = jnp.dot(h, wl_ref[...], preferred_element_type=jnp.float32)
  r2_ref[...] = jnp.dot(h, wr_ref[...], preferred_element_type=jnp.float32)


def _tc_c_body(p_ref, r_ref, b_ref, o_ref):
  sb = p_ref[0] + p_ref[1] + r_ref[...] + b_ref[...][None, :]
  o_ref[...] = jnp.where(sb > 0, sb, jnp.exp(sb) - 1.0)


_row_spec = pl.BlockSpec((ROW_BLK, D), lambda i: (i, 0))
_p_spec = pl.BlockSpec((NC, ROW_BLK, D), lambda i: (0, i, 0))
_w_spec = pl.BlockSpec((D, D), lambda i: (0, 0))
_b_spec = pl.BlockSpec((D,), lambda i: (0,))

_tc_a = pl.pallas_call(
    _tc_a_body,
    grid=(N // ROW_BLK,),
    in_specs=[_row_spec, _w_spec, _w_spec],
    out_specs=[_row_spec, _row_spec],
    out_shape=[jax.ShapeDtypeStruct((N, D), jnp.float32)] * 2,
)

_tc_b = pl.pallas_call(
    _tc_b_body,
    grid=(N // ROW_BLK,),
    in_specs=[_p_spec, _row_spec, _b_spec, _w_spec, _w_spec],
    out_specs=[_row_spec, _row_spec],
    out_shape=[jax.ShapeDtypeStruct((N, D), jnp.float32)] * 2,
)

_tc_c = pl.pallas_call(
    _tc_c_body,
    grid=(N // ROW_BLK,),
    in_specs=[_p_spec, _row_spec, _b_spec],
    out_specs=_row_spec,
    out_shape=jax.ShapeDtypeStruct((N, D), jnp.float32),
)


def kernel(x, edge_index, Wl1, bl1, Wr1, Wl2, bl2, Wr2):
  src = edge_index[0].astype(jnp.int32)
  dst = edge_index[1].astype(jnp.int32)
  npad = E_PAD - E
  src_p = jnp.concatenate([src, jnp.zeros((npad,), jnp.int32)])
  dst_p = jnp.concatenate(
      [dst, N + (jnp.arange(npad, dtype=jnp.int32) % (N_PAD - N))])
  src2 = src_p.reshape(E_PAD // B_E, B_E)
  dst2 = dst_p.reshape(E_PAD // B_E, B_E)

  y1, r1 = _tc_a(x, Wl1, Wr1)
  p1, inv = _sc_agg1()(y1, src2, dst2)
  y2, r2 = _tc_b(p1, r1, bl1, Wl2, Wr2)
  p2 = _sc_agg2()(y2, src2, dst2, inv)
  return _tc_c(p2, r2, bl2)


# speedup vs baseline: 1.0293x; 1.0293x over previous
import functools

import jax
import jax.numpy as jnp
from jax import lax
from jax.experimental import pallas as pl
from jax.experimental.pallas import tpu as pltpu
from jax.experimental.pallas import tpu_sc as plsc

N = 10000
E = 320000
D = 128
L = 16
NC = 2
NS = 16
N_PAD = 10240
ROWS_PER_TILE = N_PAD // NS
E_PAD = 327680
E_TILE = E_PAD // (NC * NS)
B_E = 128
N_EBATCH = E_TILE // B_E
NBUF = 2
CORE0_BATCHES = 112
CORE1_BATCHES = 2 * N_EBATCH - CORE0_BATCHES
CHUNK_R = 16
E_CNT_TILE = E_PAD // NS
CNT_ROWS = E_CNT_TILE // B_E
CNT_CHUNK_ROWS = 16
CROWS = N_PAD // B_E
CROWS_PER_TILE = CROWS // NS
ZROWS = 16
ROW_BLK = 1000

_mesh = functools.partial(
    plsc.VectorSubcoreMesh,
    core_axis_name="c", subcore_axis_name="s", num_cores=NC, num_subcores=NS)


def _zero_rows(ref, nrows):
  zero16 = jnp.zeros((L,), jnp.float32)
  def row(i, _):
    def col(q, _):
      ref[i, pl.ds(q * L, L)] = zero16
      return 0
    return lax.fori_loop(0, D // L, col, 0)
  lax.fori_loop(0, nrows, row, 0)


def _edge_scatter_loop(y_hbm, src2_hbm, dst2_hbm, acc_sh, src2_v, dst2_v,
                       rows_bufs, sems, c, s):
  nb = jnp.where(c == 0, CORE0_BATCHES, CORE1_BATCHES)
  row0 = c * (NS * CORE0_BATCHES) + s * nb
  def chunk_loop(ch, _):
    cr0 = row0 + ch * CHUNK_R
    pltpu.sync_copy(src2_hbm.at[pl.ds(cr0, CHUNK_R)], src2_v)
    pltpu.sync_copy(dst2_hbm.at[pl.ds(cr0, CHUNK_R)], dst2_v)
    for b in range(NBUF):
      pltpu.async_copy(y_hbm.at[src2_v.at[b]], rows_bufs[b], sems[b])
    def step(g0, _):
      for b in range(NBUF):
        g = g0 * NBUF + b
        pltpu.make_async_copy(
            y_hbm.at[src2_v.at[g]], rows_bufs[b], sems[b]).wait()
        pltpu.sync_copy(rows_bufs[b], acc_sh.at[dst2_v.at[g]], add=True)
        pltpu.async_copy(y_hbm.at[src2_v.at[g + NBUF]], rows_bufs[b], sems[b])
      return 0
    lax.fori_loop(0, CHUNK_R // NBUF - 1, step, 0)
    for b in range(NBUF):
      g = CHUNK_R - NBUF + b
      pltpu.make_async_copy(
          y_hbm.at[src2_v.at[g]], rows_bufs[b], sems[b]).wait()
      pltpu.sync_copy(rows_bufs[b], acc_sh.at[dst2_v.at[g]], add=True)
    return 0
  lax.fori_loop(0, nb // CHUNK_R, chunk_loop, 0)


def _scale_and_emit(acc_sh, inv_v, stage_v, p_hbm, c, r0):
  def chunk(j, _):
    pltpu.sync_copy(acc_sh.at[pl.ds(r0 + j * ZROWS, ZROWS)], stage_v)
    def row(rr, _):
      idx16 = jnp.full((L,), j * ZROWS + rr, jnp.int32)
      g = plsc.load_gather(inv_v, [idx16])
      def col(q, _):
        stage_v[rr, pl.ds(q * L, L)] = stage_v[rr, pl.ds(q * L, L)] * g
        return 0
      return lax.fori_loop(0, D // L, col, 0)
    lax.fori_loop(0, ZROWS, row, 0)
    pltpu.sync_copy(stage_v, p_hbm.at[c, pl.ds(r0 + j * ZROWS, ZROWS)])
    return 0
  lax.fori_loop(0, ROWS_PER_TILE // ZROWS, chunk, 0)


def _zero_acc(acc_sh, stage_v, r0):
  def zacc(j, _):
    pltpu.sync_copy(stage_v, acc_sh.at[pl.ds(r0 + j * ZROWS, ZROWS)])
    return 0
  lax.fori_loop(0, ROWS_PER_TILE // ZROWS, zacc, 0)


def _sc_agg1_body(y_hbm, src2_hbm, dst2_hbm, p_hbm, inv_hbm,
                  acc_sh, cnt_sh, src2_v, dst2_v, cnt5_v, inv_v,
                  sem0, sem1):
  c = lax.axis_index("c")
  s = lax.axis_index("s")
  r0 = s * ROWS_PER_TILE
  zero16 = jnp.zeros((L,), jnp.float32)
  ones16 = jnp.full((L,), 1.0, jnp.float32)

  def phase_a(stage_v, cntloc_v, dstbuf_v, iota_v):
    _zero_rows(stage_v, ZROWS)
    def zcnt(i, _):
      def zcntc(q, _):
        cntloc_v[i, pl.ds(q * L, L)] = zero16
        return 0
      return lax.fori_loop(0, B_E // L, zcntc, 0)
    lax.fori_loop(0, CROWS, zcnt, 0)
    _zero_acc(acc_sh, stage_v, r0)
    pltpu.sync_copy(stage_v.at[pl.ds(0, CROWS_PER_TILE)],
                    cnt_sh.at[pl.ds(s * CROWS_PER_TILE, CROWS_PER_TILE)])

    t0r = s * CNT_ROWS
    def cnt_chunk(jc, _):
      pltpu.sync_copy(
          dst2_hbm.at[pl.ds(t0r + jc * CNT_CHUNK_ROWS, CNT_CHUNK_ROWS)],
          dstbuf_v)
      def cnt_row(rr, _):
        def cnt_col(q, _):
          d16 = dstbuf_v[rr, pl.ds(q * L, L)]
          row16 = lax.shift_right_logical(d16, 7)
          col16 = lax.bitwise_and(d16, B_E - 1)
          plsc.addupdate_scatter(cntloc_v, [row16, col16], ones16)
          return 0
        return lax.fori_loop(0, B_E // L, cnt_col, 0)
      return lax.fori_loop(0, CNT_CHUNK_ROWS, cnt_row, 0)
    lax.fori_loop(0, CNT_ROWS // CNT_CHUNK_ROWS, cnt_chunk, 0)

    def mkiota(k, _):
      iota_v[pl.ds(k * L, L)] = lax.iota(jnp.int32, L) + k * L
      return 0
    lax.fori_loop(0, CROWS // L, mkiota, 0)

    plsc.subcore_barrier()
    pltpu.sync_copy(cntloc_v, cnt_sh.at[iota_v], add=True)

  pl.run_scoped(
      phase_a,
      pltpu.VMEM((ZROWS, D), jnp.float32),
      pltpu.VMEM((CROWS, B_E), jnp.float32),
      pltpu.VMEM((CNT_CHUNK_ROWS, B_E), jnp.int32),
      pltpu.VMEM((CROWS,), jnp.int32),
  )

  def phase_b(rows0, rows1):
    _edge_scatter_loop(y_hbm, src2_hbm, dst2_hbm, acc_sh, src2_v, dst2_v,
                       (rows0, rows1), (sem0, sem1), c, s)

  pl.run_scoped(phase_b,
                pltpu.VMEM((B_E, D), jnp.float32),
                pltpu.VMEM((B_E, D), jnp.float32))

  plsc.subcore_barrier()

  pltpu.sync_copy(cnt_sh.at[pl.ds(s * CROWS_PER_TILE, CROWS_PER_TILE)],
                  cnt5_v)
  def invrow(j, _):
    def invcol(q, _):
      v = cnt5_v[j, pl.ds(q * L, L)]
      inv_v[pl.ds((j * (B_E // L) + q) * L, L)] = ones16 / jnp.maximum(v, ones16)
      return 0
    return lax.fori_loop(0, B_E // L, invcol, 0)
  lax.fori_loop(0, CROWS_PER_TILE, invrow, 0)

  @pl.when(c == 0)
  def _():
    pltpu.sync_copy(inv_v, inv_hbm.at[pl.ds(r0, ROWS_PER_TILE)])

  def phase_c(stage_v):
    _scale_and_emit(acc_sh, inv_v, stage_v, p_hbm, c, r0)
  pl.run_scoped(phase_c, pltpu.VMEM((ZROWS, D), jnp.float32))


def _sc_agg2_body(y_hbm, src2_hbm, dst2_hbm, inv_hbm, p_hbm,
                  acc_sh, src2_v, dst2_v, stage_v, inv_v,
                  rows0, rows1, sem0, sem1):
  c = lax.axis_index("c")
  s = lax.axis_index("s")
  r0 = s * ROWS_PER_TILE

  _zero_rows(stage_v, ZROWS)
  _zero_acc(acc_sh, stage_v, r0)
  pltpu.sync_copy(inv_hbm.at[pl.ds(r0, ROWS_PER_TILE)], inv_v)

  plsc.subcore_barrier()

  _edge_scatter_loop(y_hbm, src2_hbm, dst2_hbm, acc_sh, src2_v, dst2_v,
                     (rows0, rows1), (sem0, sem1), c, s)

  plsc.subcore_barrier()

  _scale_and_emit(acc_sh, inv_v, stage_v, p_hbm, c, r0)


_ROWBUFS = [pltpu.VMEM((B_E, D), jnp.float32)] * NBUF
_SEMS = [pltpu.SemaphoreType.DMA] * NBUF


@functools.lru_cache(maxsize=None)
def _sc_agg1():
  return pl.kernel(
    _sc_agg1_body,
    out_type=(jax.ShapeDtypeStruct((NC, N_PAD, D), jnp.float32),
              jax.ShapeDtypeStruct((N_PAD,), jnp.float32)),
    mesh=_mesh(),
    compiler_params=pltpu.CompilerParams(needs_layout_passes=False),
    scratch_types=[
        pltpu.VMEM_SHARED((N_PAD, D), jnp.float32),
        pltpu.VMEM_SHARED((CROWS, B_E), jnp.float32),
        pltpu.VMEM((CHUNK_R, B_E), jnp.int32),
        pltpu.VMEM((CHUNK_R, B_E), jnp.int32),
        pltpu.VMEM((CROWS_PER_TILE, B_E), jnp.float32),
        pltpu.VMEM((ROWS_PER_TILE,), jnp.float32),
        *_SEMS,
    ],
  )


@functools.lru_cache(maxsize=None)
def _sc_agg2():
  return pl.kernel(
    _sc_agg2_body,
    out_type=jax.ShapeDtypeStruct((NC, N_PAD, D), jnp.float32),
    mesh=_mesh(),
    compiler_params=pltpu.CompilerParams(needs_layout_passes=False),
    scratch_types=[
        pltpu.VMEM_SHARED((N_PAD, D), jnp.float32),
        pltpu.VMEM((CHUNK_R, B_E), jnp.int32),
        pltpu.VMEM((CHUNK_R, B_E), jnp.int32),
        pltpu.VMEM((ZROWS, D), jnp.float32),
        pltpu.VMEM((ROWS_PER_TILE,), jnp.float32),
        *_ROWBUFS,
        *_SEMS,
    ],
  )


def _tc_a_body(x_ref, wl_ref, wr_ref, y_ref, r_ref):
  xb = x_ref[...]
  y_ref[...] = jnp.dot(xb, wl_ref[...], preferred_element_type=jnp.float32)
  r_ref[...] = jnp.dot(xb, wr_ref[...], preferred_element_type=jnp.float32)


def _tc_b_body(p_ref, r_ref, b_ref, wl_ref, wr_ref, y2_ref, r2_ref):
  sb = p_ref[0] + p_ref[1] + r_ref[...] + b_ref[...][None, :]
  h = jnp.where(sb > 0, sb, jnp.exp(sb) - 1.0)
  y2_ref[...] = jnp.dot(h, wl_ref[...], preferred_element_type=jnp.float32)
  r2_ref[...] = jnp.dot(h, wr_ref[...], preferred_element_type=jnp.float32)


def _tc_c_body(p_ref, r_ref, b_ref, o_ref):
  sb = p_ref[0] + p_ref[1] + r_ref[...] + b_ref[...][None, :]
  o_ref[...] = jnp.where(sb > 0, sb, jnp.exp(sb) - 1.0)


_row_spec = pl.BlockSpec((ROW_BLK, D), lambda i: (i, 0))
_p_spec = pl.BlockSpec((NC, ROW_BLK, D), lambda i: (0, i, 0))
_w_spec = pl.BlockSpec((D, D), lambda i: (0, 0))
_b_spec = pl.BlockSpec((D,), lambda i: (0,))

_tc_a = pl.pallas_call(
    _tc_a_body,
    grid=(N // ROW_BLK,),
    in_specs=[_row_spec, _w_spec, _w_spec],
    out_specs=[_row_spec, _row_spec],
    out_shape=[jax.ShapeDtypeStruct((N, D), jnp.float32)] * 2,
)

_tc_b = pl.pallas_call(
    _tc_b_body,
    grid=(N // ROW_BLK,),
    in_specs=[_p_spec, _row_spec, _b_spec, _w_spec, _w_spec],
    out_specs=[_row_spec, _row_spec],
    out_shape=[jax.ShapeDtypeStruct((N, D), jnp.float32)] * 2,
)

_tc_c = pl.pallas_call(
    _tc_c_body,
    grid=(N // ROW_BLK,),
    in_specs=[_p_spec, _row_spec, _b_spec],
    out_specs=_row_spec,
    out_shape=jax.ShapeDtypeStruct((N, D), jnp.float32),
)


def kernel(x, edge_index, Wl1, bl1, Wr1, Wl2, bl2, Wr2):
  src = edge_index[0].astype(jnp.int32)
  dst = edge_index[1].astype(jnp.int32)
  npad = E_PAD - E
  src_p = jnp.concatenate([src, jnp.zeros((npad,), jnp.int32)])
  dst_p = jnp.concatenate(
      [dst, N + (jnp.arange(npad, dtype=jnp.int32) % (N_PAD - N))])
  src2 = src_p.reshape(E_PAD // B_E, B_E)
  dst2 = dst_p.reshape(E_PAD // B_E, B_E)

  y1, r1 = _tc_a(x, Wl1, Wr1)
  p1, inv = _sc_agg1()(y1, src2, dst2)
  y2, r2 = _tc_b(p1, r1, bl1, Wl2, Wr2)
  p2 = _sc_agg2()(y2, src2, dst2, inv)
  return _tc_c(p2, r2, bl2)


# speedup vs baseline: 3.2027x; 3.1115x over previous
import functools

import jax
import jax.numpy as jnp
from jax import lax
from jax.experimental import pallas as pl
from jax.experimental.pallas import tpu as pltpu
from jax.experimental.pallas import tpu_sc as plsc

N = 10000
E = 320000
D = 128
L = 16
NC = 2
NS = 16
N_PAD = 10240
ROWS_PER_TILE = N_PAD // NS
E_PAD = 327680
E_TILE = E_PAD // (NC * NS)
B_E = 128
N_EBATCH = E_TILE // B_E
NBUF = 2
CORE0_BATCHES = 80
CORE1_BATCHES = 2 * N_EBATCH - CORE0_BATCHES
CHUNK_R = 16
E_CNT_TILE = E_PAD // NS
CNT_ROWS = E_CNT_TILE // B_E
CNT_CHUNK_ROWS = 16
CROWS = N_PAD // B_E
CROWS_PER_TILE = CROWS // NS
ZROWS = 16
ROW_BLK = 1000

_mesh = functools.partial(
    plsc.VectorSubcoreMesh,
    core_axis_name="c", subcore_axis_name="s", num_cores=NC, num_subcores=NS)


def _zero_rows(ref, nrows):
  zero16 = jnp.zeros((L,), jnp.float32)
  def row(i, _):
    def col(q, _):
      ref[i, pl.ds(q * L, L)] = zero16
      return 0
    return lax.fori_loop(0, D // L, col, 0)
  lax.fori_loop(0, nrows, row, 0)


def _edge_scatter_loop(y_hbm, src2_hbm, dst2_hbm, acc_sh, src2_v, dst2_v,
                       rows_bufs, sems, c, s):
  nb = jnp.where(c == 0, CORE0_BATCHES, CORE1_BATCHES)
  row0 = c * (NS * CORE0_BATCHES) + s * nb
  def chunk_loop(ch, _):
    cr0 = row0 + ch * CHUNK_R
    pltpu.sync_copy(src2_hbm.at[pl.ds(cr0, CHUNK_R)], src2_v)
    pltpu.sync_copy(dst2_hbm.at[pl.ds(cr0, CHUNK_R)], dst2_v)
    for b in range(NBUF):
      pltpu.async_copy(y_hbm.at[src2_v.at[b]], rows_bufs[b], sems[b])
    def step(g0, _):
      for b in range(NBUF):
        g = g0 * NBUF + b
        pltpu.make_async_copy(
            y_hbm.at[src2_v.at[g]], rows_bufs[b], sems[b]).wait()
        pltpu.sync_copy(rows_bufs[b], acc_sh.at[dst2_v.at[g]], add=True)
        pltpu.async_copy(y_hbm.at[src2_v.at[g + NBUF]], rows_bufs[b], sems[b])
      return 0
    lax.fori_loop(0, CHUNK_R // NBUF - 1, step, 0)
    for b in range(NBUF):
      g = CHUNK_R - NBUF + b
      pltpu.make_async_copy(
          y_hbm.at[src2_v.at[g]], rows_bufs[b], sems[b]).wait()
      pltpu.sync_copy(rows_bufs[b], acc_sh.at[dst2_v.at[g]], add=True)
    return 0
  lax.fori_loop(0, nb // CHUNK_R, chunk_loop, 0)


def _scale_and_emit(acc_sh, inv_v, stage_v, p_hbm, c, r0):
  def chunk(j, _):
    pltpu.sync_copy(acc_sh.at[pl.ds(r0 + j * ZROWS, ZROWS)], stage_v)
    def row(rr, _):
      idx16 = jnp.full((L,), j * ZROWS + rr, jnp.int32)
      g = plsc.load_gather(inv_v, [idx16])
      def col(q, _):
        stage_v[rr, pl.ds(q * L, L)] = stage_v[rr, pl.ds(q * L, L)] * g
        return 0
      return lax.fori_loop(0, D // L, col, 0)
    lax.fori_loop(0, ZROWS, row, 0)
    pltpu.sync_copy(stage_v, p_hbm.at[c, pl.ds(r0 + j * ZROWS, ZROWS)])
    return 0
  lax.fori_loop(0, ROWS_PER_TILE // ZROWS, chunk, 0)


def _zero_acc(acc_sh, stage_v, r0):
  def zacc(j, _):
    pltpu.sync_copy(stage_v, acc_sh.at[pl.ds(r0 + j * ZROWS, ZROWS)])
    return 0
  lax.fori_loop(0, ROWS_PER_TILE // ZROWS, zacc, 0)


def _sc_agg1_body(y_hbm, src2_hbm, dst2_hbm, p_hbm, inv_hbm,
                  acc_sh, cnt_sh, src2_v, dst2_v, cnt5_v, inv_v,
                  sem0, sem1):
  c = lax.axis_index("c")
  s = lax.axis_index("s")
  r0 = s * ROWS_PER_TILE
  zero16 = jnp.zeros((L,), jnp.float32)
  ones16 = jnp.full((L,), 1.0, jnp.float32)

  def phase_a(stage_v, cntloc_v, dstbuf_v, iota_v):
    _zero_rows(stage_v, ZROWS)
    def zcnt(i, _):
      def zcntc(q, _):
        cntloc_v[i, pl.ds(q * L, L)] = zero16
        return 0
      return lax.fori_loop(0, B_E // L, zcntc, 0)
    lax.fori_loop(0, CROWS, zcnt, 0)
    _zero_acc(acc_sh, stage_v, r0)
    pltpu.sync_copy(stage_v.at[pl.ds(0, CROWS_PER_TILE)],
                    cnt_sh.at[pl.ds(s * CROWS_PER_TILE, CROWS_PER_TILE)])

    t0r = s * CNT_ROWS
    def cnt_chunk(jc, _):
      pltpu.sync_copy(
          dst2_hbm.at[pl.ds(t0r + jc * CNT_CHUNK_ROWS, CNT_CHUNK_ROWS)],
          dstbuf_v)
      def cnt_row(rr, _):
        def cnt_col(q, _):
          d16 = dstbuf_v[rr, pl.ds(q * L, L)]
          row16 = lax.shift_right_logical(d16, 7)
          col16 = lax.bitwise_and(d16, B_E - 1)
          plsc.addupdate_scatter(cntloc_v, [row16, col16], ones16)
          return 0
        return lax.fori_loop(0, B_E // L, cnt_col, 0)
      return lax.fori_loop(0, CNT_CHUNK_ROWS, cnt_row, 0)
    lax.fori_loop(0, CNT_ROWS // CNT_CHUNK_ROWS, cnt_chunk, 0)

    def mkiota(k, _):
      iota_v[pl.ds(k * L, L)] = lax.iota(jnp.int32, L) + k * L
      return 0
    lax.fori_loop(0, CROWS // L, mkiota, 0)

    plsc.subcore_barrier()
    pltpu.sync_copy(cntloc_v, cnt_sh.at[iota_v], add=True)

  pl.run_scoped(
      phase_a,
      pltpu.VMEM((ZROWS, D), jnp.float32),
      pltpu.VMEM((CROWS, B_E), jnp.float32),
      pltpu.VMEM((CNT_CHUNK_ROWS, B_E), jnp.int32),
      pltpu.VMEM((CROWS,), jnp.int32),
  )

  def phase_b(rows0, rows1):
    _edge_scatter_loop(y_hbm, src2_hbm, dst2_hbm, acc_sh, src2_v, dst2_v,
                       (rows0, rows1), (sem0, sem1), c, s)

  pl.run_scoped(phase_b,
                pltpu.VMEM((B_E, D), jnp.float32),
                pltpu.VMEM((B_E, D), jnp.float32))

  plsc.subcore_barrier()

  pltpu.sync_copy(cnt_sh.at[pl.ds(s * CROWS_PER_TILE, CROWS_PER_TILE)],
                  cnt5_v)
  def invrow(j, _):
    def invcol(q, _):
      v = cnt5_v[j, pl.ds(q * L, L)]
      inv_v[pl.ds((j * (B_E // L) + q) * L, L)] = ones16 / jnp.maximum(v, ones16)
      return 0
    return lax.fori_loop(0, B_E // L, invcol, 0)
  lax.fori_loop(0, CROWS_PER_TILE, invrow, 0)

  @pl.when(c == 0)
  def _():
    pltpu.sync_copy(inv_v, inv_hbm.at[pl.ds(r0, ROWS_PER_TILE)])

  def phase_c(stage_v):
    _scale_and_emit(acc_sh, inv_v, stage_v, p_hbm, c, r0)
  pl.run_scoped(phase_c, pltpu.VMEM((ZROWS, D), jnp.float32))


def _sc_agg2_body(y_hbm, src2_hbm, dst2_hbm, inv_hbm, p_hbm,
                  acc_sh, src2_v, dst2_v, stage_v, inv_v,
                  rows0, rows1, sem0, sem1):
  c = lax.axis_index("c")
  s = lax.axis_index("s")
  r0 = s * ROWS_PER_TILE

  _zero_rows(stage_v, ZROWS)
  _zero_acc(acc_sh, stage_v, r0)
  pltpu.sync_copy(inv_hbm.at[pl.ds(r0, ROWS_PER_TILE)], inv_v)

  plsc.subcore_barrier()

  _edge_scatter_loop(y_hbm, src2_hbm, dst2_hbm, acc_sh, src2_v, dst2_v,
                     (rows0, rows1), (sem0, sem1), c, s)

  plsc.subcore_barrier()

  _scale_and_emit(acc_sh, inv_v, stage_v, p_hbm, c, r0)


_ROWBUFS = [pltpu.VMEM((B_E, D), jnp.float32)] * NBUF
_SEMS = [pltpu.SemaphoreType.DMA] * NBUF


@functools.lru_cache(maxsize=None)
def _sc_agg1():
  return pl.kernel(
    _sc_agg1_body,
    out_type=(jax.ShapeDtypeStruct((NC, N_PAD, D), jnp.float32),
              jax.ShapeDtypeStruct((N_PAD,), jnp.float32)),
    mesh=_mesh(),
    compiler_params=pltpu.CompilerParams(needs_layout_passes=False),
    scratch_types=[
        pltpu.VMEM_SHARED((N_PAD, D), jnp.float32),
        pltpu.VMEM_SHARED((CROWS, B_E), jnp.float32),
        pltpu.VMEM((CHUNK_R, B_E), jnp.int32),
        pltpu.VMEM((CHUNK_R, B_E), jnp.int32),
        pltpu.VMEM((CROWS_PER_TILE, B_E), jnp.float32),
        pltpu.VMEM((ROWS_PER_TILE,), jnp.float32),
        *_SEMS,
    ],
  )


@functools.lru_cache(maxsize=None)
def _sc_agg2():
  return pl.kernel(
    _sc_agg2_body,
    out_type=jax.ShapeDtypeStruct((NC, N_PAD, D), jnp.float32),
    mesh=_mesh(),
    compiler_params=pltpu.CompilerParams(needs_layout_passes=False),
    scratch_types=[
        pltpu.VMEM_SHARED((N_PAD, D), jnp.float32),
        pltpu.VMEM((CHUNK_R, B_E), jnp.int32),
        pltpu.VMEM((CHUNK_R, B_E), jnp.int32),
        pltpu.VMEM((ZROWS, D), jnp.float32),
        pltpu.VMEM((ROWS_PER_TILE,), jnp.float32),
        *_ROWBUFS,
        *_SEMS,
    ],
  )


def _tc_a_body(x_ref, wl_ref, wr_ref, y_ref, r_ref):
  xb = x_ref[...]
  y_ref[...] = jnp.dot(xb, wl_ref[...], preferred_element_type=jnp.float32)
  r_ref[...] = jnp.dot(xb, wr_ref[...], preferred_element_type=jnp.float32)


def _tc_b_body(p_ref, r_ref, b_ref, wl_ref, wr_ref, y2_ref, r2_ref):
  sb = p_ref[0] + p_ref[1] + r_ref[...] + b_ref[...][None, :]
  h = jnp.where(sb > 0, sb, jnp.exp(sb) - 1.0)
  y2_ref[...] = jnp.dot(h, wl_ref[...], preferred_element_type=jnp.float32)
  r2_ref[...] = jnp.dot(h, wr_ref[...], preferred_element_type=jnp.float32)


def _tc_c_body(p_ref, r_ref, b_ref, o_ref):
  sb = p_ref[0] + p_ref[1] + r_ref[...] + b_ref[...][None, :]
  o_ref[...] = jnp.where(sb > 0, sb, jnp.exp(sb) - 1.0)


_row_spec = pl.BlockSpec((ROW_BLK, D), lambda i: (i, 0))
_p_spec = pl.BlockSpec((NC, ROW_BLK, D), lambda i: (0, i, 0))
_w_spec = pl.BlockSpec((D, D), lambda i: (0, 0))
_b_spec = pl.BlockSpec((D,), lambda i: (0,))

_tc_a = pl.pallas_call(
    _tc_a_body,
    grid=(N // ROW_BLK,),
    in_specs=[_row_spec, _w_spec, _w_spec],
    out_specs=[_row_spec, _row_spec],
    out_shape=[jax.ShapeDtypeStruct((N, D), jnp.float32)] * 2,
)

_tc_b = pl.pallas_call(
    _tc_b_body,
    grid=(N // ROW_BLK,),
    in_specs=[_p_spec, _row_spec, _b_spec, _w_spec, _w_spec],
    out_specs=[_row_spec, _row_spec],
    out_shape=[jax.ShapeDtypeStruct((N, D), jnp.float32)] * 2,
)

_tc_c = pl.pallas_call(
    _tc_c_body,
    grid=(N // ROW_BLK,),
    in_specs=[_p_spec, _row_spec, _b_spec],
    out_specs=_row_spec,
    out_shape=jax.ShapeDtypeStruct((N, D), jnp.float32),
)


def kernel(x, edge_index, Wl1, bl1, Wr1, Wl2, bl2, Wr2):
  src = edge_index[0].astype(jnp.int32)
  dst = edge_index[1].astype(jnp.int32)
  npad = E_PAD - E
  src_p = jnp.concatenate([src, jnp.arange(npad, dtype=jnp.int32) % N])
  dst_p = jnp.concatenate(
      [dst, N + (jnp.arange(npad, dtype=jnp.int32) % (N_PAD - N))])
  src2 = src_p.reshape(E_PAD // B_E, B_E)
  dst2 = dst_p.reshape(E_PAD // B_E, B_E)

  y1, r1 = _tc_a(x, Wl1, Wr1)
  p1, inv = _sc_agg1()(y1, src2, dst2)
  y2, r2 = _tc_b(p1, r1, bl1, Wl2, Wr2)
  p2 = _sc_agg2()(y2, src2, dst2, inv)
  return _tc_c(p2, r2, bl2)


# speedup vs baseline: 3.4442x; 1.0754x over previous
import functools

import jax
import jax.numpy as jnp
from jax import lax
from jax.experimental import pallas as pl
from jax.experimental.pallas import tpu as pltpu
from jax.experimental.pallas import tpu_sc as plsc

N = 10000
E = 320000
D = 128
L = 16
NC = 2
NS = 16
N_PAD = 10240
ROWS_PER_TILE = N_PAD // NS
E_PAD = 327680
E_TILE = E_PAD // (NC * NS)
B_E = 128
N_EBATCH = E_TILE // B_E
NBUF = 2
CORE0_BATCHES = 80
CORE1_BATCHES = 2 * N_EBATCH - CORE0_BATCHES
CHUNK_R = 40
E_CNT_TILE = E_PAD // NS
CNT_ROWS = E_CNT_TILE // B_E
CNT_CHUNK_ROWS = 16
CROWS = N_PAD // B_E
CROWS_PER_TILE = CROWS // NS
ZROWS = 32
ROW_BLK = 1000

_mesh = functools.partial(
    plsc.VectorSubcoreMesh,
    core_axis_name="c", subcore_axis_name="s", num_cores=NC, num_subcores=NS)


def _zero_rows(ref, nrows):
  zero16 = jnp.zeros((L,), jnp.float32)
  def row(i, _):
    def col(q, _):
      ref[i, pl.ds(q * L, L)] = zero16
      return 0
    return lax.fori_loop(0, D // L, col, 0)
  lax.fori_loop(0, nrows, row, 0)


def _edge_scatter_loop(y_hbm, src2_hbm, dst2_hbm, acc_sh, src2_v, dst2_v,
                       rows_bufs, sems, c, s):
  nb = jnp.where(c == 0, CORE0_BATCHES, CORE1_BATCHES)
  row0 = c * (NS * CORE0_BATCHES) + s * nb
  def chunk_loop(ch, _):
    cr0 = row0 + ch * CHUNK_R
    pltpu.sync_copy(src2_hbm.at[pl.ds(cr0, CHUNK_R)], src2_v)
    pltpu.sync_copy(dst2_hbm.at[pl.ds(cr0, CHUNK_R)], dst2_v)
    for b in range(NBUF):
      pltpu.async_copy(y_hbm.at[src2_v.at[b]], rows_bufs[b], sems[b])
    def step(g0, _):
      for b in range(NBUF):
        g = g0 * NBUF + b
        pltpu.make_async_copy(
            y_hbm.at[src2_v.at[g]], rows_bufs[b], sems[b]).wait()
        pltpu.sync_copy(rows_bufs[b], acc_sh.at[dst2_v.at[g]], add=True)
        pltpu.async_copy(y_hbm.at[src2_v.at[g + NBUF]], rows_bufs[b], sems[b])
      return 0
    lax.fori_loop(0, CHUNK_R // NBUF - 1, step, 0)
    for b in range(NBUF):
      g = CHUNK_R - NBUF + b
      pltpu.make_async_copy(
          y_hbm.at[src2_v.at[g]], rows_bufs[b], sems[b]).wait()
      pltpu.sync_copy(rows_bufs[b], acc_sh.at[dst2_v.at[g]], add=True)
    return 0
  lax.fori_loop(0, nb // CHUNK_R, chunk_loop, 0)


def _scale_and_emit(acc_sh, inv_v, stage_v, p_hbm, c, r0):
  def chunk(j, _):
    pltpu.sync_copy(acc_sh.at[pl.ds(r0 + j * ZROWS, ZROWS)], stage_v)
    def row(rr, _):
      idx16 = jnp.full((L,), j * ZROWS + rr, jnp.int32)
      g = plsc.load_gather(inv_v, [idx16])
      def col(q, _):
        stage_v[rr, pl.ds(q * L, L)] = stage_v[rr, pl.ds(q * L, L)] * g
        return 0
      return lax.fori_loop(0, D // L, col, 0)
    lax.fori_loop(0, ZROWS, row, 0)
    pltpu.sync_copy(stage_v, p_hbm.at[c, pl.ds(r0 + j * ZROWS, ZROWS)])
    return 0
  lax.fori_loop(0, ROWS_PER_TILE // ZROWS, chunk, 0)


def _zero_acc(acc_sh, stage_v, r0):
  def zacc(j, _):
    pltpu.sync_copy(stage_v, acc_sh.at[pl.ds(r0 + j * ZROWS, ZROWS)])
    return 0
  lax.fori_loop(0, ROWS_PER_TILE // ZROWS, zacc, 0)


def _sc_agg1_body(y_hbm, src2_hbm, dst2_hbm, p_hbm, inv_hbm,
                  acc_sh, cnt_sh, src2_v, dst2_v, cnt5_v, inv_v,
                  sem0, sem1):
  c = lax.axis_index("c")
  s = lax.axis_index("s")
  r0 = s * ROWS_PER_TILE
  zero16 = jnp.zeros((L,), jnp.float32)
  ones16 = jnp.full((L,), 1.0, jnp.float32)

  def phase_a(stage_v, cntloc_v, dstbuf_v, iota_v):
    _zero_rows(stage_v, ZROWS)
    def zcnt(i, _):
      def zcntc(q, _):
        cntloc_v[i, pl.ds(q * L, L)] = zero16
        return 0
      return lax.fori_loop(0, B_E // L, zcntc, 0)
    lax.fori_loop(0, CROWS, zcnt, 0)
    _zero_acc(acc_sh, stage_v, r0)
    pltpu.sync_copy(stage_v.at[pl.ds(0, CROWS_PER_TILE)],
                    cnt_sh.at[pl.ds(s * CROWS_PER_TILE, CROWS_PER_TILE)])

    t0r = s * CNT_ROWS
    def cnt_chunk(jc, _):
      pltpu.sync_copy(
          dst2_hbm.at[pl.ds(t0r + jc * CNT_CHUNK_ROWS, CNT_CHUNK_ROWS)],
          dstbuf_v)
      def cnt_row(rr, _):
        def cnt_col(q, _):
          d16 = dstbuf_v[rr, pl.ds(q * L, L)]
          row16 = lax.shift_right_logical(d16, 7)
          col16 = lax.bitwise_and(d16, B_E - 1)
          plsc.addupdate_scatter(cntloc_v, [row16, col16], ones16)
          return 0
        return lax.fori_loop(0, B_E // L, cnt_col, 0)
      return lax.fori_loop(0, CNT_CHUNK_ROWS, cnt_row, 0)
    lax.fori_loop(0, CNT_ROWS // CNT_CHUNK_ROWS, cnt_chunk, 0)

    def mkiota(k, _):
      iota_v[pl.ds(k * L, L)] = lax.iota(jnp.int32, L) + k * L
      return 0
    lax.fori_loop(0, CROWS // L, mkiota, 0)

    plsc.subcore_barrier()
    pltpu.sync_copy(cntloc_v, cnt_sh.at[iota_v], add=True)

  pl.run_scoped(
      phase_a,
      pltpu.VMEM((ZROWS, D), jnp.float32),
      pltpu.VMEM((CROWS, B_E), jnp.float32),
      pltpu.VMEM((CNT_CHUNK_ROWS, B_E), jnp.int32),
      pltpu.VMEM((CROWS,), jnp.int32),
  )

  def phase_b(rows0, rows1):
    _edge_scatter_loop(y_hbm, src2_hbm, dst2_hbm, acc_sh, src2_v, dst2_v,
                       (rows0, rows1), (sem0, sem1), c, s)

  pl.run_scoped(phase_b,
                pltpu.VMEM((B_E, D), jnp.float32),
                pltpu.VMEM((B_E, D), jnp.float32))

  plsc.subcore_barrier()

  pltpu.sync_copy(cnt_sh.at[pl.ds(s * CROWS_PER_TILE, CROWS_PER_TILE)],
                  cnt5_v)
  def invrow(j, _):
    def invcol(q, _):
      v = cnt5_v[j, pl.ds(q * L, L)]
      inv_v[pl.ds((j * (B_E // L) + q) * L, L)] = ones16 / jnp.maximum(v, ones16)
      return 0
    return lax.fori_loop(0, B_E // L, invcol, 0)
  lax.fori_loop(0, CROWS_PER_TILE, invrow, 0)

  @pl.when(c == 0)
  def _():
    pltpu.sync_copy(inv_v, inv_hbm.at[pl.ds(r0, ROWS_PER_TILE)])

  def phase_c(stage_v):
    _scale_and_emit(acc_sh, inv_v, stage_v, p_hbm, c, r0)
  pl.run_scoped(phase_c, pltpu.VMEM((ZROWS, D), jnp.float32))


def _sc_agg2_body(y_hbm, src2_hbm, dst2_hbm, inv_hbm, p_hbm,
                  acc_sh, src2_v, dst2_v, stage_v, inv_v,
                  rows0, rows1, sem0, sem1):
  c = lax.axis_index("c")
  s = lax.axis_index("s")
  r0 = s * ROWS_PER_TILE

  _zero_rows(stage_v, ZROWS)
  _zero_acc(acc_sh, stage_v, r0)
  pltpu.sync_copy(inv_hbm.at[pl.ds(r0, ROWS_PER_TILE)], inv_v)

  plsc.subcore_barrier()

  _edge_scatter_loop(y_hbm, src2_hbm, dst2_hbm, acc_sh, src2_v, dst2_v,
                     (rows0, rows1), (sem0, sem1), c, s)

  plsc.subcore_barrier()

  _scale_and_emit(acc_sh, inv_v, stage_v, p_hbm, c, r0)


_ROWBUFS = [pltpu.VMEM((B_E, D), jnp.float32)] * NBUF
_SEMS = [pltpu.SemaphoreType.DMA] * NBUF


@functools.lru_cache(maxsize=None)
def _sc_agg1():
  return pl.kernel(
    _sc_agg1_body,
    out_type=(jax.ShapeDtypeStruct((NC, N_PAD, D), jnp.float32),
              jax.ShapeDtypeStruct((N_PAD,), jnp.float32)),
    mesh=_mesh(),
    compiler_params=pltpu.CompilerParams(needs_layout_passes=False),
    scratch_types=[
        pltpu.VMEM_SHARED((N_PAD, D), jnp.float32),
        pltpu.VMEM_SHARED((CROWS, B_E), jnp.float32),
        pltpu.VMEM((CHUNK_R, B_E), jnp.int32),
        pltpu.VMEM((CHUNK_R, B_E), jnp.int32),
        pltpu.VMEM((CROWS_PER_TILE, B_E), jnp.float32),
        pltpu.VMEM((ROWS_PER_TILE,), jnp.float32),
        *_SEMS,
    ],
  )


@functools.lru_cache(maxsize=None)
def _sc_agg2():
  return pl.kernel(
    _sc_agg2_body,
    out_type=jax.ShapeDtypeStruct((NC, N_PAD, D), jnp.float32),
    mesh=_mesh(),
    compiler_params=pltpu.CompilerParams(needs_layout_passes=False),
    scratch_types=[
        pltpu.VMEM_SHARED((N_PAD, D), jnp.float32),
        pltpu.VMEM((CHUNK_R, B_E), jnp.int32),
        pltpu.VMEM((CHUNK_R, B_E), jnp.int32),
        pltpu.VMEM((ZROWS, D), jnp.float32),
        pltpu.VMEM((ROWS_PER_TILE,), jnp.float32),
        *_ROWBUFS,
        *_SEMS,
    ],
  )


def _tc_a_body(x_ref, wl_ref, wr_ref, y_ref, r_ref):
  xb = x_ref[...]
  y_ref[...] = jnp.dot(xb, wl_ref[...], preferred_element_type=jnp.float32)
  r_ref[...] = jnp.dot(xb, wr_ref[...], preferred_element_type=jnp.float32)


def _tc_b_body(p_ref, r_ref, b_ref, wl_ref, wr_ref, y2_ref, r2_ref):
  sb = p_ref[0] + p_ref[1] + r_ref[...] + b_ref[...][None, :]
  h = jnp.where(sb > 0, sb, jnp.exp(sb) - 1.0)
  y2_ref[...] = jnp.dot(h, wl_ref[...], preferred_element_type=jnp.float32)
  r2_ref[...] = jnp.dot(h, wr_ref[...], preferred_element_type=jnp.float32)


def _tc_c_body(p_ref, r_ref, b_ref, o_ref):
  sb = p_ref[0] + p_ref[1] + r_ref[...] + b_ref[...][None, :]
  o_ref[...] = jnp.where(sb > 0, sb, jnp.exp(sb) - 1.0)


_row_spec = pl.BlockSpec((ROW_BLK, D), lambda i: (i, 0))
_p_spec = pl.BlockSpec((NC, ROW_BLK, D), lambda i: (0, i, 0))
_w_spec = pl.BlockSpec((D, D), lambda i: (0, 0))
_b_spec = pl.BlockSpec((D,), lambda i: (0,))

_tc_a = pl.pallas_call(
    _tc_a_body,
    grid=(N // ROW_BLK,),
    in_specs=[_row_spec, _w_spec, _w_spec],
    out_specs=[_row_spec, _row_spec],
    out_shape=[jax.ShapeDtypeStruct((N, D), jnp.float32)] * 2,
)

_tc_b = pl.pallas_call(
    _tc_b_body,
    grid=(N // ROW_BLK,),
    in_specs=[_p_spec, _row_spec, _b_spec, _w_spec, _w_spec],
    out_specs=[_row_spec, _row_spec],
    out_shape=[jax.ShapeDtypeStruct((N, D), jnp.float32)] * 2,
)

_tc_c = pl.pallas_call(
    _tc_c_body,
    grid=(N // ROW_BLK,),
    in_specs=[_p_spec, _row_spec, _b_spec],
    out_specs=_row_spec,
    out_shape=jax.ShapeDtypeStruct((N, D), jnp.float32),
)


def kernel(x, edge_index, Wl1, bl1, Wr1, Wl2, bl2, Wr2):
  src = edge_index[0].astype(jnp.int32)
  dst = edge_index[1].astype(jnp.int32)
  npad = E_PAD - E
  src_p = jnp.concatenate([src, jnp.arange(npad, dtype=jnp.int32) % N])
  dst_p = jnp.concatenate(
      [dst, N + (jnp.arange(npad, dtype=jnp.int32) % (N_PAD - N))])
  src2 = src_p.reshape(E_PAD // B_E, B_E)
  dst2 = dst_p.reshape(E_PAD // B_E, B_E)

  y1, r1 = _tc_a(x, Wl1, Wr1)
  p1, inv = _sc_agg1()(y1, src2, dst2)
  y2, r2 = _tc_b(p1, r1, bl1, Wl2, Wr2)
  p2 = _sc_agg2()(y2, src2, dst2, inv)
  return _tc_c(p2, r2, bl2)


# speedup vs baseline: 3.5640x; 1.0348x over previous
import functools

import jax
import jax.numpy as jnp
from jax import lax
from jax.experimental import pallas as pl
from jax.experimental.pallas import tpu as pltpu
from jax.experimental.pallas import tpu_sc as plsc

N = 10000
E = 320000
D = 128
L = 16
NC = 2
NS = 16
N_PAD = 10240
ROWS_PER_TILE = N_PAD // NS
E_PAD = 327680
E_TILE = E_PAD // (NC * NS)
B_E = 128
N_EBATCH = E_TILE // B_E
NBUF = 2
CORE0_BATCHES = 80
CORE1_BATCHES = 2 * N_EBATCH - CORE0_BATCHES
CHUNK_R = 40
E_CNT_TILE = E_PAD // NS
CNT_ROWS = E_CNT_TILE // B_E
CNT_CHUNK_ROWS = 16
CROWS = N_PAD // B_E
CROWS_PER_TILE = CROWS // NS
ZROWS = 32
ROW_BLK = 1000

_mesh = functools.partial(
    plsc.VectorSubcoreMesh,
    core_axis_name="c", subcore_axis_name="s", num_cores=NC, num_subcores=NS)


def _zero_rows(ref, nrows):
  zero16 = jnp.zeros((L,), jnp.float32)
  def row(i, _):
    def col(q, _):
      ref[i, pl.ds(q * L, L)] = zero16
      return 0
    return lax.fori_loop(0, D // L, col, 0)
  lax.fori_loop(0, nrows, row, 0)


def _edge_scatter_loop(y_hbm, src2_hbm, dst2_hbm, acc_sh, src2_v, dst2_v,
                       rows_bufs, sems, c, s):
  nb = jnp.where(c == 0, CORE0_BATCHES, CORE1_BATCHES)
  row0 = c * (NS * CORE0_BATCHES) + s * nb
  def chunk_loop(ch, _):
    cr0 = row0 + ch * CHUNK_R
    pltpu.sync_copy(src2_hbm.at[pl.ds(cr0, CHUNK_R)], src2_v)
    pltpu.sync_copy(dst2_hbm.at[pl.ds(cr0, CHUNK_R)], dst2_v)
    for b in range(NBUF):
      pltpu.async_copy(y_hbm.at[src2_v.at[b]], rows_bufs[b], sems[b])
    def step(g0, _):
      for b in range(NBUF):
        g = g0 * NBUF + b
        pltpu.make_async_copy(
            y_hbm.at[src2_v.at[g]], rows_bufs[b], sems[b]).wait()
        pltpu.sync_copy(rows_bufs[b], acc_sh.at[dst2_v.at[g]], add=True)
        pltpu.async_copy(y_hbm.at[src2_v.at[g + NBUF]], rows_bufs[b], sems[b])
      return 0
    lax.fori_loop(0, CHUNK_R // NBUF - 1, step, 0)
    for b in range(NBUF):
      g = CHUNK_R - NBUF + b
      pltpu.make_async_copy(
          y_hbm.at[src2_v.at[g]], rows_bufs[b], sems[b]).wait()
      pltpu.sync_copy(rows_bufs[b], acc_sh.at[dst2_v.at[g]], add=True)
    return 0
  lax.fori_loop(0, nb // CHUNK_R, chunk_loop, 0)


def _scale_and_emit(acc_sh, inv_v, stages, p_hbm, c, r0, sems_i, sems_o):
  nch = ROWS_PER_TILE // ZROWS
  for b in range(2):
    pltpu.async_copy(acc_sh.at[pl.ds(r0 + b * ZROWS, ZROWS)], stages[b],
                     sems_i[b])
  def step(j0, _):
    for b in range(2):
      j = j0 * 2 + b
      pltpu.make_async_copy(acc_sh.at[pl.ds(r0 + j * ZROWS, ZROWS)],
                            stages[b], sems_i[b]).wait()
      def row(rr, _):
        idx16 = jnp.full((L,), j * ZROWS + rr, jnp.int32)
        g = plsc.load_gather(inv_v, [idx16])
        def col(q, _):
          stages[b][rr, pl.ds(q * L, L)] = stages[b][rr, pl.ds(q * L, L)] * g
          return 0
        return lax.fori_loop(0, D // L, col, 0)
      lax.fori_loop(0, ZROWS, row, 0)
      pltpu.async_copy(stages[b], p_hbm.at[c, pl.ds(r0 + j * ZROWS, ZROWS)],
                       sems_o[b])
      @pl.when(j + 2 < nch)
      def _():
        pltpu.make_async_copy(
            stages[b], p_hbm.at[c, pl.ds(r0 + j * ZROWS, ZROWS)],
            sems_o[b]).wait()
        pltpu.async_copy(acc_sh.at[pl.ds(r0 + (j + 2) * ZROWS, ZROWS)],
                         stages[b], sems_i[b])
    return 0
  lax.fori_loop(0, nch // 2, step, 0)
  for b in range(2):
    j = nch - 2 + b
    pltpu.make_async_copy(
        stages[b], p_hbm.at[c, pl.ds(r0 + j * ZROWS, ZROWS)],
        sems_o[b]).wait()


def _zero_acc(acc_sh, stage_v, r0, sem):
  nch = ROWS_PER_TILE // ZROWS
  def fire(j, _):
    pltpu.async_copy(stage_v, acc_sh.at[pl.ds(r0 + j * ZROWS, ZROWS)], sem)
    return 0
  lax.fori_loop(0, nch, fire, 0)
  def drain(j, _):
    pltpu.make_async_copy(stage_v, acc_sh.at[pl.ds(r0 + j * ZROWS, ZROWS)],
                          sem).wait()
    return 0
  lax.fori_loop(0, nch, drain, 0)


def _sc_agg1_body(y_hbm, src2_hbm, dst2_hbm, p_hbm, inv_hbm,
                  acc_sh, cnt_sh, src2_v, dst2_v, cnt5_v, inv_v,
                  sem0, sem1, sem2, sem3):
  c = lax.axis_index("c")
  s = lax.axis_index("s")
  r0 = s * ROWS_PER_TILE
  zero16 = jnp.zeros((L,), jnp.float32)
  ones16 = jnp.full((L,), 1.0, jnp.float32)

  def phase_a(stage_v, cntloc_v, dstbuf_v, iota_v):
    _zero_rows(stage_v, ZROWS)
    def zcnt(i, _):
      def zcntc(q, _):
        cntloc_v[i, pl.ds(q * L, L)] = zero16
        return 0
      return lax.fori_loop(0, B_E // L, zcntc, 0)
    lax.fori_loop(0, CROWS, zcnt, 0)
    _zero_acc(acc_sh, stage_v, r0, sem0)
    pltpu.sync_copy(stage_v.at[pl.ds(0, CROWS_PER_TILE)],
                    cnt_sh.at[pl.ds(s * CROWS_PER_TILE, CROWS_PER_TILE)])

    t0r = s * CNT_ROWS
    def cnt_chunk(jc, _):
      pltpu.sync_copy(
          dst2_hbm.at[pl.ds(t0r + jc * CNT_CHUNK_ROWS, CNT_CHUNK_ROWS)],
          dstbuf_v)
      def cnt_row(rr, _):
        def cnt_col(q, _):
          d16 = dstbuf_v[rr, pl.ds(q * L, L)]
          row16 = lax.shift_right_logical(d16, 7)
          col16 = lax.bitwise_and(d16, B_E - 1)
          plsc.addupdate_scatter(cntloc_v, [row16, col16], ones16)
          return 0
        return lax.fori_loop(0, B_E // L, cnt_col, 0)
      return lax.fori_loop(0, CNT_CHUNK_ROWS, cnt_row, 0)
    lax.fori_loop(0, CNT_ROWS // CNT_CHUNK_ROWS, cnt_chunk, 0)

    def mkiota(k, _):
      iota_v[pl.ds(k * L, L)] = lax.iota(jnp.int32, L) + k * L
      return 0
    lax.fori_loop(0, CROWS // L, mkiota, 0)

    plsc.subcore_barrier()
    pltpu.sync_copy(cntloc_v, cnt_sh.at[iota_v], add=True)

  pl.run_scoped(
      phase_a,
      pltpu.VMEM((ZROWS, D), jnp.float32),
      pltpu.VMEM((CROWS, B_E), jnp.float32),
      pltpu.VMEM((CNT_CHUNK_ROWS, B_E), jnp.int32),
      pltpu.VMEM((CROWS,), jnp.int32),
  )

  def phase_b(rows0, rows1):
    _edge_scatter_loop(y_hbm, src2_hbm, dst2_hbm, acc_sh, src2_v, dst2_v,
                       (rows0, rows1), (sem0, sem1), c, s)

  pl.run_scoped(phase_b,
                pltpu.VMEM((B_E, D), jnp.float32),
                pltpu.VMEM((B_E, D), jnp.float32))

  plsc.subcore_barrier()

  pltpu.sync_copy(cnt_sh.at[pl.ds(s * CROWS_PER_TILE, CROWS_PER_TILE)],
                  cnt5_v)
  def invrow(j, _):
    def invcol(q, _):
      v = cnt5_v[j, pl.ds(q * L, L)]
      inv_v[pl.ds((j * (B_E // L) + q) * L, L)] = ones16 / jnp.maximum(v, ones16)
      return 0
    return lax.fori_loop(0, B_E // L, invcol, 0)
  lax.fori_loop(0, CROWS_PER_TILE, invrow, 0)

  @pl.when(c == 0)
  def _():
    pltpu.sync_copy(inv_v, inv_hbm.at[pl.ds(r0, ROWS_PER_TILE)])

  def phase_c(st0, st1):
    _scale_and_emit(acc_sh, inv_v, (st0, st1), p_hbm, c, r0,
                    (sem0, sem1), (sem2, sem3))
  pl.run_scoped(phase_c, pltpu.VMEM((ZROWS, D), jnp.float32),
                pltpu.VMEM((ZROWS, D), jnp.float32))


def _sc_agg2_body(y_hbm, src2_hbm, dst2_hbm, inv_hbm, p_hbm,
                  acc_sh, src2_v, dst2_v, inv_v,
                  sem0, sem1, sem2, sem3):
  c = lax.axis_index("c")
  s = lax.axis_index("s")
  r0 = s * ROWS_PER_TILE

  def phase_a(stage_v):
    _zero_rows(stage_v, ZROWS)
    _zero_acc(acc_sh, stage_v, r0, sem0)
  pl.run_scoped(phase_a, pltpu.VMEM((ZROWS, D), jnp.float32))
  pltpu.sync_copy(inv_hbm.at[pl.ds(r0, ROWS_PER_TILE)], inv_v)

  plsc.subcore_barrier()

  def phase_b(rows0, rows1):
    _edge_scatter_loop(y_hbm, src2_hbm, dst2_hbm, acc_sh, src2_v, dst2_v,
                       (rows0, rows1), (sem0, sem1), c, s)
  pl.run_scoped(phase_b,
                pltpu.VMEM((B_E, D), jnp.float32),
                pltpu.VMEM((B_E, D), jnp.float32))

  plsc.subcore_barrier()

  def phase_c(st0, st1):
    _scale_and_emit(acc_sh, inv_v, (st0, st1), p_hbm, c, r0,
                    (sem0, sem1), (sem2, sem3))
  pl.run_scoped(phase_c, pltpu.VMEM((ZROWS, D), jnp.float32),
                pltpu.VMEM((ZROWS, D), jnp.float32))


_SEMS = [pltpu.SemaphoreType.DMA] * 4


@functools.lru_cache(maxsize=None)
def _sc_agg1():
  return pl.kernel(
    _sc_agg1_body,
    out_type=(jax.ShapeDtypeStruct((NC, N_PAD, D), jnp.float32),
              jax.ShapeDtypeStruct((N_PAD,), jnp.float32)),
    mesh=_mesh(),
    compiler_params=pltpu.CompilerParams(needs_layout_passes=False),
    scratch_types=[
        pltpu.VMEM_SHARED((N_PAD, D), jnp.float32),
        pltpu.VMEM_SHARED((CROWS, B_E), jnp.float32),
        pltpu.VMEM((CHUNK_R, B_E), jnp.int32),
        pltpu.VMEM((CHUNK_R, B_E), jnp.int32),
        pltpu.VMEM((CROWS_PER_TILE, B_E), jnp.float32),
        pltpu.VMEM((ROWS_PER_TILE,), jnp.float32),
        *_SEMS,
    ],
  )


@functools.lru_cache(maxsize=None)
def _sc_agg2():
  return pl.kernel(
    _sc_agg2_body,
    out_type=jax.ShapeDtypeStruct((NC, N_PAD, D), jnp.float32),
    mesh=_mesh(),
    compiler_params=pltpu.CompilerParams(needs_layout_passes=False),
    scratch_types=[
        pltpu.VMEM_SHARED((N_PAD, D), jnp.float32),
        pltpu.VMEM((CHUNK_R, B_E), jnp.int32),
        pltpu.VMEM((CHUNK_R, B_E), jnp.int32),
        pltpu.VMEM((ROWS_PER_TILE,), jnp.float32),
        *_SEMS,
    ],
  )


def _tc_a_body(x_ref, wl_ref, wr_ref, y_ref, r_ref):
  xb = x_ref[...]
  y_ref[...] = jnp.dot(xb, wl_ref[...], preferred_element_type=jnp.float32)
  r_ref[...] = jnp.dot(xb, wr_ref[...], preferred_element_type=jnp.float32)


def _tc_b_body(p_ref, r_ref, b_ref, wl_ref, wr_ref, y2_ref, r2_ref):
  sb = p_ref[0] + p_ref[1] + r_ref[...] + b_ref[...][None, :]
  h = jnp.where(sb > 0, sb, jnp.exp(sb) - 1.0)
  y2_ref[...] = jnp.dot(h, wl_ref[...], preferred_element_type=jnp.float32)
  r2_ref[...] = jnp.dot(h, wr_ref[...], preferred_element_type=jnp.float32)


def _tc_c_body(p_ref, r_ref, b_ref, o_ref):
  sb = p_ref[0] + p_ref[1] + r_ref[...] + b_ref[...][None, :]
  o_ref[...] = jnp.where(sb > 0, sb, jnp.exp(sb) - 1.0)


_row_spec = pl.BlockSpec((ROW_BLK, D), lambda i: (i, 0))
_p_spec = pl.BlockSpec((NC, ROW_BLK, D), lambda i: (0, i, 0))
_w_spec = pl.BlockSpec((D, D), lambda i: (0, 0))
_b_spec = pl.BlockSpec((D,), lambda i: (0,))

_tc_a = pl.pallas_call(
    _tc_a_body,
    grid=(N // ROW_BLK,),
    in_specs=[_row_spec, _w_spec, _w_spec],
    out_specs=[_row_spec, _row_spec],
    out_shape=[jax.ShapeDtypeStruct((N, D), jnp.float32)] * 2,
)

_tc_b = pl.pallas_call(
    _tc_b_body,
    grid=(N // ROW_BLK,),
    in_specs=[_p_spec, _row_spec, _b_spec, _w_spec, _w_spec],
    out_specs=[_row_spec, _row_spec],
    out_shape=[jax.ShapeDtypeStruct((N, D), jnp.float32)] * 2,
)

_tc_c = pl.pallas_call(
    _tc_c_body,
    grid=(N // ROW_BLK,),
    in_specs=[_p_spec, _row_spec, _b_spec],
    out_specs=_row_spec,
    out_shape=jax.ShapeDtypeStruct((N, D), jnp.float32),
)


def kernel(x, edge_index, Wl1, bl1, Wr1, Wl2, bl2, Wr2):
  src = edge_index[0].astype(jnp.int32)
  dst = edge_index[1].astype(jnp.int32)
  npad = E_PAD - E
  src_p = jnp.concatenate([src, jnp.arange(npad, dtype=jnp.int32) % N])
  dst_p = jnp.concatenate(
      [dst, N + (jnp.arange(npad, dtype=jnp.int32) % (N_PAD - N))])
  src2 = src_p.reshape(E_PAD // B_E, B_E)
  dst2 = dst_p.reshape(E_PAD // B_E, B_E)

  y1, r1 = _tc_a(x, Wl1, Wr1)
  p1, inv = _sc_agg1()(y1, src2, dst2)
  y2, r2 = _tc_b(p1, r1, bl1, Wl2, Wr2)
  p2 = _sc_agg2()(y2, src2, dst2, inv)
  return _tc_c(p2, r2, bl2)


# speedup vs baseline: 3.7118x; 1.0415x over previous
import functools

import jax
import jax.numpy as jnp
from jax import lax
from jax.experimental import pallas as pl
from jax.experimental.pallas import tpu as pltpu
from jax.experimental.pallas import tpu_sc as plsc

N = 10000
E = 320000
D = 128
L = 16
NC = 2
NS = 16
N_PAD = 10240
ROWS_PER_TILE = N_PAD // NS
E_PAD = 327680
E_TILE = E_PAD // (NC * NS)
B_E = 128
N_EBATCH = E_TILE // B_E
NBUF = 2
CORE0_BATCHES = 80
CORE1_BATCHES = 2 * N_EBATCH - CORE0_BATCHES
CHUNK_R = 40
E_CNT_TILE = E_PAD // NS
CNT_ROWS = E_CNT_TILE // B_E
CNT_CHUNK_ROWS = 16
CROWS = N_PAD // B_E
CROWS_PER_TILE = CROWS // NS
ZROWS = 32
ROW_BLK = 1000
NPAD_ROWS = (E_PAD - E) // B_E

_mesh = functools.partial(
    plsc.VectorSubcoreMesh,
    core_axis_name="c", subcore_axis_name="s", num_cores=NC, num_subcores=NS)


def _zero_rows(ref, nrows):
  zero16 = jnp.zeros((L,), jnp.float32)
  def row(i, _):
    def col(q, _):
      ref[i, pl.ds(q * L, L)] = zero16
      return 0
    return lax.fori_loop(0, D // L, col, 0)
  lax.fori_loop(0, nrows, row, 0)


def _edge_scatter_loop(y_hbm, src2_hbm, dst2_hbm, acc_sh, src2_v, dst2_v,
                       rows_bufs, sems, c, s):
  nb = jnp.where(c == 0, CORE0_BATCHES, CORE1_BATCHES)
  row0 = c * (NS * CORE0_BATCHES) + s * nb
  def chunk_loop(ch, _):
    cr0 = row0 + ch * CHUNK_R
    pltpu.sync_copy(src2_hbm.at[pl.ds(cr0, CHUNK_R)], src2_v)
    pltpu.sync_copy(dst2_hbm.at[pl.ds(cr0, CHUNK_R)], dst2_v)
    for b in range(NBUF):
      pltpu.async_copy(y_hbm.at[src2_v.at[b]], rows_bufs[b], sems[b])
    def step(g0, _):
      for b in range(NBUF):
        g = g0 * NBUF + b
        pltpu.make_async_copy(
            y_hbm.at[src2_v.at[g]], rows_bufs[b], sems[b]).wait()
        pltpu.sync_copy(rows_bufs[b], acc_sh.at[dst2_v.at[g]], add=True)
        pltpu.async_copy(y_hbm.at[src2_v.at[g + NBUF]], rows_bufs[b], sems[b])
      return 0
    lax.fori_loop(0, CHUNK_R // NBUF - 1, step, 0)
    for b in range(NBUF):
      g = CHUNK_R - NBUF + b
      pltpu.make_async_copy(
          y_hbm.at[src2_v.at[g]], rows_bufs[b], sems[b]).wait()
      pltpu.sync_copy(rows_bufs[b], acc_sh.at[dst2_v.at[g]], add=True)
    return 0
  lax.fori_loop(0, nb // CHUNK_R, chunk_loop, 0)


def _scale_and_emit(acc_sh, inv_v, stages, p_hbm, c, r0, sems_i, sems_o):
  nch = ROWS_PER_TILE // ZROWS
  for b in range(2):
    pltpu.async_copy(acc_sh.at[pl.ds(r0 + b * ZROWS, ZROWS)], stages[b],
                     sems_i[b])
  def step(j0, _):
    for b in range(2):
      j = j0 * 2 + b
      pltpu.make_async_copy(acc_sh.at[pl.ds(r0 + j * ZROWS, ZROWS)],
                            stages[b], sems_i[b]).wait()
      def row(rr, _):
        idx16 = jnp.full((L,), j * ZROWS + rr, jnp.int32)
        g = plsc.load_gather(inv_v, [idx16])
        def col(q, _):
          stages[b][rr, pl.ds(q * L, L)] = stages[b][rr, pl.ds(q * L, L)] * g
          return 0
        return lax.fori_loop(0, D // L, col, 0)
      lax.fori_loop(0, ZROWS, row, 0)
      pltpu.async_copy(stages[b], p_hbm.at[c, pl.ds(r0 + j * ZROWS, ZROWS)],
                       sems_o[b])
      @pl.when(j + 2 < nch)
      def _():
        pltpu.make_async_copy(
            stages[b], p_hbm.at[c, pl.ds(r0 + j * ZROWS, ZROWS)],
            sems_o[b]).wait()
        pltpu.async_copy(acc_sh.at[pl.ds(r0 + (j + 2) * ZROWS, ZROWS)],
                         stages[b], sems_i[b])
    return 0
  lax.fori_loop(0, nch // 2, step, 0)
  for b in range(2):
    j = nch - 2 + b
    pltpu.make_async_copy(
        stages[b], p_hbm.at[c, pl.ds(r0 + j * ZROWS, ZROWS)],
        sems_o[b]).wait()


def _zero_acc(acc_sh, stage_v, r0, sem):
  nch = ROWS_PER_TILE // ZROWS
  def fire(j, _):
    pltpu.async_copy(stage_v, acc_sh.at[pl.ds(r0 + j * ZROWS, ZROWS)], sem)
    return 0
  lax.fori_loop(0, nch, fire, 0)
  def drain(j, _):
    pltpu.make_async_copy(stage_v, acc_sh.at[pl.ds(r0 + j * ZROWS, ZROWS)],
                          sem).wait()
    return 0
  lax.fori_loop(0, nch, drain, 0)


def _sc_agg1_body(y_hbm, src2_hbm, dst2_hbm, p_hbm, inv_hbm,
                  acc_sh, cnt_sh, src2_v, dst2_v, cnt5_v, inv_v,
                  sem0, sem1, sem2, sem3):
  c = lax.axis_index("c")
  s = lax.axis_index("s")
  r0 = s * ROWS_PER_TILE
  zero16 = jnp.zeros((L,), jnp.float32)
  ones16 = jnp.full((L,), 1.0, jnp.float32)

  def phase_a(stage_v, cntloc_v, dstbuf_v, iota_v):
    _zero_rows(stage_v, ZROWS)
    def zcnt(i, _):
      def zcntc(q, _):
        cntloc_v[i, pl.ds(q * L, L)] = zero16
        return 0
      return lax.fori_loop(0, B_E // L, zcntc, 0)
    lax.fori_loop(0, CROWS, zcnt, 0)
    _zero_acc(acc_sh, stage_v, r0, sem0)
    pltpu.sync_copy(stage_v.at[pl.ds(0, CROWS_PER_TILE)],
                    cnt_sh.at[pl.ds(s * CROWS_PER_TILE, CROWS_PER_TILE)])

    t0r = s * CNT_ROWS
    def cnt_chunk(jc, _):
      pltpu.sync_copy(
          dst2_hbm.at[pl.ds(t0r + jc * CNT_CHUNK_ROWS, CNT_CHUNK_ROWS)],
          dstbuf_v)
      def cnt_row(rr, _):
        def cnt_col(q, _):
          d16 = dstbuf_v[rr, pl.ds(q * L, L)]
          row16 = lax.shift_right_logical(d16, 7)
          col16 = lax.bitwise_and(d16, B_E - 1)
          plsc.addupdate_scatter(cntloc_v, [row16, col16], ones16)
          return 0
        return lax.fori_loop(0, B_E // L, cnt_col, 0)
      return lax.fori_loop(0, CNT_CHUNK_ROWS, cnt_row, 0)
    lax.fori_loop(0, CNT_ROWS // CNT_CHUNK_ROWS, cnt_chunk, 0)

    def mkiota(k, _):
      iota_v[pl.ds(k * L, L)] = lax.iota(jnp.int32, L) + k * L
      return 0
    lax.fori_loop(0, CROWS // L, mkiota, 0)

    plsc.subcore_barrier()
    pltpu.sync_copy(cntloc_v, cnt_sh.at[iota_v], add=True)

  pl.run_scoped(
      phase_a,
      pltpu.VMEM((ZROWS, D), jnp.float32),
      pltpu.VMEM((CROWS, B_E), jnp.float32),
      pltpu.VMEM((CNT_CHUNK_ROWS, B_E), jnp.int32),
      pltpu.VMEM((CROWS,), jnp.int32),
  )

  def phase_b(rows0, rows1):
    _edge_scatter_loop(y_hbm, src2_hbm, dst2_hbm, acc_sh, src2_v, dst2_v,
                       (rows0, rows1), (sem0, sem1), c, s)

  pl.run_scoped(phase_b,
                pltpu.VMEM((B_E, D), jnp.float32),
                pltpu.VMEM((B_E, D), jnp.float32))

  plsc.subcore_barrier()

  pltpu.sync_copy(cnt_sh.at[pl.ds(s * CROWS_PER_TILE, CROWS_PER_TILE)],
                  cnt5_v)
  def invrow(j, _):
    def invcol(q, _):
      v = cnt5_v[j, pl.ds(q * L, L)]
      inv_v[pl.ds((j * (B_E // L) + q) * L, L)] = ones16 / jnp.maximum(v, ones16)
      return 0
    return lax.fori_loop(0, B_E // L, invcol, 0)
  lax.fori_loop(0, CROWS_PER_TILE, invrow, 0)

  @pl.when(c == 0)
  def _():
    pltpu.sync_copy(inv_v, inv_hbm.at[pl.ds(r0, ROWS_PER_TILE)])

  def phase_c(st0, st1):
    _scale_and_emit(acc_sh, inv_v, (st0, st1), p_hbm, c, r0,
                    (sem0, sem1), (sem2, sem3))
  pl.run_scoped(phase_c, pltpu.VMEM((ZROWS, D), jnp.float32),
                pltpu.VMEM((ZROWS, D), jnp.float32))


def _sc_agg2_body(y_hbm, src2_hbm, dst2_hbm, inv_hbm, p_hbm,
                  acc_sh, src2_v, dst2_v, inv_v,
                  sem0, sem1, sem2, sem3):
  c = lax.axis_index("c")
  s = lax.axis_index("s")
  r0 = s * ROWS_PER_TILE

  def phase_a(stage_v):
    _zero_rows(stage_v, ZROWS)
    _zero_acc(acc_sh, stage_v, r0, sem0)
  pl.run_scoped(phase_a, pltpu.VMEM((ZROWS, D), jnp.float32))
  pltpu.sync_copy(inv_hbm.at[pl.ds(r0, ROWS_PER_TILE)], inv_v)

  plsc.subcore_barrier()

  def phase_b(rows0, rows1):
    _edge_scatter_loop(y_hbm, src2_hbm, dst2_hbm, acc_sh, src2_v, dst2_v,
                       (rows0, rows1), (sem0, sem1), c, s)
  pl.run_scoped(phase_b,
                pltpu.VMEM((B_E, D), jnp.float32),
                pltpu.VMEM((B_E, D), jnp.float32))

  plsc.subcore_barrier()

  def phase_c(st0, st1):
    _scale_and_emit(acc_sh, inv_v, (st0, st1), p_hbm, c, r0,
                    (sem0, sem1), (sem2, sem3))
  pl.run_scoped(phase_c, pltpu.VMEM((ZROWS, D), jnp.float32),
                pltpu.VMEM((ZROWS, D), jnp.float32))


_SEMS = [pltpu.SemaphoreType.DMA] * 4


@functools.lru_cache(maxsize=None)
def _sc_agg1():
  return pl.kernel(
    _sc_agg1_body,
    out_type=(jax.ShapeDtypeStruct((NC, N_PAD, D), jnp.float32),
              jax.ShapeDtypeStruct((N_PAD,), jnp.float32)),
    mesh=_mesh(),
    compiler_params=pltpu.CompilerParams(needs_layout_passes=False),
    scratch_types=[
        pltpu.VMEM_SHARED((N_PAD, D), jnp.float32),
        pltpu.VMEM_SHARED((CROWS, B_E), jnp.float32),
        pltpu.VMEM((CHUNK_R, B_E), jnp.int32),
        pltpu.VMEM((CHUNK_R, B_E), jnp.int32),
        pltpu.VMEM((CROWS_PER_TILE, B_E), jnp.float32),
        pltpu.VMEM((ROWS_PER_TILE,), jnp.float32),
        *_SEMS,
    ],
  )


@functools.lru_cache(maxsize=None)
def _sc_agg2():
  return pl.kernel(
    _sc_agg2_body,
    out_type=jax.ShapeDtypeStruct((NC, N_PAD, D), jnp.float32),
    mesh=_mesh(),
    compiler_params=pltpu.CompilerParams(needs_layout_passes=False),
    scratch_types=[
        pltpu.VMEM_SHARED((N_PAD, D), jnp.float32),
        pltpu.VMEM((CHUNK_R, B_E), jnp.int32),
        pltpu.VMEM((CHUNK_R, B_E), jnp.int32),
        pltpu.VMEM((ROWS_PER_TILE,), jnp.float32),
        *_SEMS,
    ],
  )


def _tc_a_body(x_ref, wl_ref, wr_ref, y_ref, r_ref):
  xb = x_ref[...]
  y_ref[...] = jnp.dot(xb, wl_ref[...], preferred_element_type=jnp.float32)
  r_ref[...] = jnp.dot(xb, wr_ref[...], preferred_element_type=jnp.float32)


def _tc_b_body(p_ref, r_ref, b_ref, wl_ref, wr_ref, y2_ref, r2_ref):
  sb = p_ref[0] + p_ref[1] + r_ref[...] + b_ref[...][None, :]
  h = jnp.where(sb > 0, sb, jnp.exp(sb) - 1.0)
  y2_ref[...] = jnp.dot(h, wl_ref[...], preferred_element_type=jnp.float32)
  r2_ref[...] = jnp.dot(h, wr_ref[...], preferred_element_type=jnp.float32)


def _tc_c_body(p_ref, r_ref, b_ref, o_ref):
  sb = p_ref[0] + p_ref[1] + r_ref[...] + b_ref[...][None, :]
  o_ref[...] = jnp.where(sb > 0, sb, jnp.exp(sb) - 1.0)


_row_spec = pl.BlockSpec((ROW_BLK, D), lambda i: (i, 0))
_p_spec = pl.BlockSpec((NC, ROW_BLK, D), lambda i: (0, i, 0))
_w_spec = pl.BlockSpec((D, D), lambda i: (0, 0))
_b_spec = pl.BlockSpec((D,), lambda i: (0,))

_tc_a = pl.pallas_call(
    _tc_a_body,
    grid=(N // ROW_BLK,),
    in_specs=[_row_spec, _w_spec, _w_spec],
    out_specs=[_row_spec, _row_spec],
    out_shape=[jax.ShapeDtypeStruct((N, D), jnp.float32)] * 2,
)

_tc_b = pl.pallas_call(
    _tc_b_body,
    grid=(N // ROW_BLK,),
    in_specs=[_p_spec, _row_spec, _b_spec, _w_spec, _w_spec],
    out_specs=[_row_spec, _row_spec],
    out_shape=[jax.ShapeDtypeStruct((N, D), jnp.float32)] * 2,
)

def _tc_prep_body(ei_ref, src2_ref, dst2_ref):
  src = ei_ref[0].reshape(E // B_E, B_E)
  dst = ei_ref[1].reshape(E // B_E, B_E)
  pad = jax.lax.broadcasted_iota(jnp.int32, (NPAD_ROWS, B_E), 0) * B_E + \
      jax.lax.broadcasted_iota(jnp.int32, (NPAD_ROWS, B_E), 1)
  src2_ref[...] = jnp.concatenate([src, pad % N], axis=0)
  dst2_ref[...] = jnp.concatenate([dst, N + pad % (N_PAD - N)], axis=0)


_tc_prep = pl.pallas_call(
    _tc_prep_body,
    in_specs=[pl.BlockSpec((2, E), lambda: (0, 0))],
    out_specs=[pl.BlockSpec((E_PAD // B_E, B_E), lambda: (0, 0))] * 2,
    out_shape=[jax.ShapeDtypeStruct((E_PAD // B_E, B_E), jnp.int32)] * 2,
)

_tc_c = pl.pallas_call(
    _tc_c_body,
    grid=(N // ROW_BLK,),
    in_specs=[_p_spec, _row_spec, _b_spec],
    out_specs=_row_spec,
    out_shape=jax.ShapeDtypeStruct((N, D), jnp.float32),
)


def kernel(x, edge_index, Wl1, bl1, Wr1, Wl2, bl2, Wr2):
  src2, dst2 = _tc_prep(edge_index.astype(jnp.int32))

  y1, r1 = _tc_a(x, Wl1, Wr1)
  p1, inv = _sc_agg1()(y1, src2, dst2)
  y2, r2 = _tc_b(p1, r1, bl1, Wl2, Wr2)
  p2 = _sc_agg2()(y2, src2, dst2, inv)
  return _tc_c(p2, r2, bl2)


# speedup vs baseline: 3.7889x; 1.0208x over previous
import functools

import jax
import jax.numpy as jnp
from jax import lax
from jax.experimental import pallas as pl
from jax.experimental.pallas import tpu as pltpu
from jax.experimental.pallas import tpu_sc as plsc

N = 10000
E = 320000
D = 128
L = 16
NC = 2
NS = 16
N_PAD = 10240
ROWS_PER_TILE = N_PAD // NS
E_PAD = 327680
E_TILE = E_PAD // (NC * NS)
B_E = 128
N_EBATCH = E_TILE // B_E
NBUF = 2
CORE0_BATCHES = 80
CORE1_BATCHES = 2 * N_EBATCH - CORE0_BATCHES
CHUNK_R = 40
E_CNT_TILE = E_PAD // NS
CNT_ROWS = E_CNT_TILE // B_E
CNT_CHUNK_ROWS = 16
CROWS = N_PAD // B_E
CROWS_PER_TILE = CROWS // NS
ZROWS = 64
ROW_BLK = 1000
NPAD_ROWS = (E_PAD - E) // B_E

_mesh = functools.partial(
    plsc.VectorSubcoreMesh,
    core_axis_name="c", subcore_axis_name="s", num_cores=NC, num_subcores=NS)


def _zero_rows(ref, nrows):
  zero16 = jnp.zeros((L,), jnp.float32)
  def row(i, _):
    def col(q, _):
      ref[i, pl.ds(q * L, L)] = zero16
      return 0
    return lax.fori_loop(0, D // L, col, 0)
  lax.fori_loop(0, nrows, row, 0)


def _edge_scatter_loop(y_hbm, src2_hbm, dst2_hbm, acc_sh, src2_v, dst2_v,
                       rows_bufs, sems, c, s):
  nb = jnp.where(c == 0, CORE0_BATCHES, CORE1_BATCHES)
  row0 = c * (NS * CORE0_BATCHES) + s * nb
  def chunk_loop(ch, _):
    cr0 = row0 + ch * CHUNK_R
    pltpu.sync_copy(src2_hbm.at[pl.ds(cr0, CHUNK_R)], src2_v)
    pltpu.sync_copy(dst2_hbm.at[pl.ds(cr0, CHUNK_R)], dst2_v)
    for b in range(NBUF):
      pltpu.async_copy(y_hbm.at[src2_v.at[b]], rows_bufs[b], sems[b])
    def step(g0, _):
      for b in range(NBUF):
        g = g0 * NBUF + b
        pltpu.make_async_copy(
            y_hbm.at[src2_v.at[g]], rows_bufs[b], sems[b]).wait()
        pltpu.sync_copy(rows_bufs[b], acc_sh.at[dst2_v.at[g]], add=True)
        pltpu.async_copy(y_hbm.at[src2_v.at[g + NBUF]], rows_bufs[b], sems[b])
      return 0
    lax.fori_loop(0, CHUNK_R // NBUF - 1, step, 0)
    for b in range(NBUF):
      g = CHUNK_R - NBUF + b
      pltpu.make_async_copy(
          y_hbm.at[src2_v.at[g]], rows_bufs[b], sems[b]).wait()
      pltpu.sync_copy(rows_bufs[b], acc_sh.at[dst2_v.at[g]], add=True)
    return 0
  lax.fori_loop(0, nb // CHUNK_R, chunk_loop, 0)


def _scale_and_emit(acc_sh, inv_v, stages, p_hbm, c, r0, sems_i, sems_o):
  nch = ROWS_PER_TILE // ZROWS
  for b in range(2):
    pltpu.async_copy(acc_sh.at[pl.ds(r0 + b * ZROWS, ZROWS)], stages[b],
                     sems_i[b])
  def step(j0, _):
    for b in range(2):
      j = j0 * 2 + b
      pltpu.make_async_copy(acc_sh.at[pl.ds(r0 + j * ZROWS, ZROWS)],
                            stages[b], sems_i[b]).wait()
      def row(rr, _):
        idx16 = jnp.full((L,), j * ZROWS + rr, jnp.int32)
        g = plsc.load_gather(inv_v, [idx16])
        def col(q, _):
          stages[b][rr, pl.ds(q * L, L)] = stages[b][rr, pl.ds(q * L, L)] * g
          return 0
        return lax.fori_loop(0, D // L, col, 0)
      lax.fori_loop(0, ZROWS, row, 0)
      pltpu.async_copy(stages[b], p_hbm.at[c, pl.ds(r0 + j * ZROWS, ZROWS)],
                       sems_o[b])
      @pl.when(j + 2 < nch)
      def _():
        pltpu.make_async_copy(
            stages[b], p_hbm.at[c, pl.ds(r0 + j * ZROWS, ZROWS)],
            sems_o[b]).wait()
        pltpu.async_copy(acc_sh.at[pl.ds(r0 + (j + 2) * ZROWS, ZROWS)],
                         stages[b], sems_i[b])
    return 0
  lax.fori_loop(0, nch // 2, step, 0)
  for b in range(2):
    j = nch - 2 + b
    pltpu.make_async_copy(
        stages[b], p_hbm.at[c, pl.ds(r0 + j * ZROWS, ZROWS)],
        sems_o[b]).wait()


def _zero_acc(acc_sh, stage_v, r0, sem):
  nch = ROWS_PER_TILE // ZROWS
  def fire(j, _):
    pltpu.async_copy(stage_v, acc_sh.at[pl.ds(r0 + j * ZROWS, ZROWS)], sem)
    return 0
  lax.fori_loop(0, nch, fire, 0)
  def drain(j, _):
    pltpu.make_async_copy(stage_v, acc_sh.at[pl.ds(r0 + j * ZROWS, ZROWS)],
                          sem).wait()
    return 0
  lax.fori_loop(0, nch, drain, 0)


def _sc_cnt_body(dst2_hbm, inv_hbm, cnt_sh, cnt5_v, inv_v):
  c = lax.axis_index("c")
  s = lax.axis_index("s")
  r0 = s * ROWS_PER_TILE
  zero16 = jnp.zeros((L,), jnp.float32)
  ones16 = jnp.full((L,), 1.0, jnp.float32)

  def scoped(cntloc_v, dstbuf_v, iota_v):
    def zcnt(i, _):
      def zcntc(q, _):
        cntloc_v[i, pl.ds(q * L, L)] = zero16
        return 0
      return lax.fori_loop(0, B_E // L, zcntc, 0)
    lax.fori_loop(0, CROWS, zcnt, 0)
    pltpu.sync_copy(cntloc_v.at[pl.ds(0, CROWS_PER_TILE)],
                    cnt_sh.at[pl.ds(s * CROWS_PER_TILE, CROWS_PER_TILE)])

    t0r = s * CNT_ROWS
    def cnt_chunk(jc, _):
      pltpu.sync_copy(
          dst2_hbm.at[pl.ds(t0r + jc * CNT_CHUNK_ROWS, CNT_CHUNK_ROWS)],
          dstbuf_v)
      def cnt_row(rr, _):
        def cnt_col(q, _):
          d16 = dstbuf_v[rr, pl.ds(q * L, L)]
          row16 = lax.shift_right_logical(d16, 7)
          col16 = lax.bitwise_and(d16, B_E - 1)
          plsc.addupdate_scatter(cntloc_v, [row16, col16], ones16)
          return 0
        return lax.fori_loop(0, B_E // L, cnt_col, 0)
      return lax.fori_loop(0, CNT_CHUNK_ROWS, cnt_row, 0)
    lax.fori_loop(0, CNT_ROWS // CNT_CHUNK_ROWS, cnt_chunk, 0)

    def mkiota(k, _):
      iota_v[pl.ds(k * L, L)] = lax.iota(jnp.int32, L) + k * L
      return 0
    lax.fori_loop(0, CROWS // L, mkiota, 0)

    plsc.subcore_barrier()
    pltpu.sync_copy(cntloc_v, cnt_sh.at[iota_v], add=True)

  pl.run_scoped(
      scoped,
      pltpu.VMEM((CROWS, B_E), jnp.float32),
      pltpu.VMEM((CNT_CHUNK_ROWS, B_E), jnp.int32),
      pltpu.VMEM((CROWS,), jnp.int32),
  )

  plsc.subcore_barrier()

  pltpu.sync_copy(cnt_sh.at[pl.ds(s * CROWS_PER_TILE, CROWS_PER_TILE)],
                  cnt5_v)
  def invrow(j, _):
    def invcol(q, _):
      v = cnt5_v[j, pl.ds(q * L, L)]
      inv_v[pl.ds((j * (B_E // L) + q) * L, L)] = ones16 / jnp.maximum(v, ones16)
      return 0
    return lax.fori_loop(0, B_E // L, invcol, 0)
  lax.fori_loop(0, CROWS_PER_TILE, invrow, 0)

  @pl.when(c == 0)
  def _():
    pltpu.sync_copy(inv_v, inv_hbm.at[pl.ds(r0, ROWS_PER_TILE)])


@functools.lru_cache(maxsize=None)
def _sc_cnt():
  return pl.kernel(
    _sc_cnt_body,
    out_type=jax.ShapeDtypeStruct((N_PAD,), jnp.float32),
    mesh=_mesh(),
    compiler_params=pltpu.CompilerParams(needs_layout_passes=False),
    scratch_types=[
        pltpu.VMEM_SHARED((CROWS, B_E), jnp.float32),
        pltpu.VMEM((CROWS_PER_TILE, B_E), jnp.float32),
        pltpu.VMEM((ROWS_PER_TILE,), jnp.float32),
    ],
  )


def _sc_agg2_body(y_hbm, src2_hbm, dst2_hbm, inv_hbm, p_hbm,
                  acc_sh, src2_v, dst2_v, inv_v,
                  sem0, sem1, sem2, sem3):
  c = lax.axis_index("c")
  s = lax.axis_index("s")
  r0 = s * ROWS_PER_TILE

  def phase_a(stage_v):
    _zero_rows(stage_v, ZROWS)
    _zero_acc(acc_sh, stage_v, r0, sem0)
  pl.run_scoped(phase_a, pltpu.VMEM((ZROWS, D), jnp.float32))
  pltpu.sync_copy(inv_hbm.at[pl.ds(r0, ROWS_PER_TILE)], inv_v)

  plsc.subcore_barrier()

  def phase_b(rows0, rows1):
    _edge_scatter_loop(y_hbm, src2_hbm, dst2_hbm, acc_sh, src2_v, dst2_v,
                       (rows0, rows1), (sem0, sem1), c, s)
  pl.run_scoped(phase_b,
                pltpu.VMEM((B_E, D), jnp.float32),
                pltpu.VMEM((B_E, D), jnp.float32))

  plsc.subcore_barrier()

  def phase_c(st0, st1):
    _scale_and_emit(acc_sh, inv_v, (st0, st1), p_hbm, c, r0,
                    (sem0, sem1), (sem2, sem3))
  pl.run_scoped(phase_c, pltpu.VMEM((ZROWS, D), jnp.float32),
                pltpu.VMEM((ZROWS, D), jnp.float32))


_SEMS = [pltpu.SemaphoreType.DMA] * 4


@functools.lru_cache(maxsize=None)
def _sc_agg2():
  return pl.kernel(
    _sc_agg2_body,
    out_type=jax.ShapeDtypeStruct((NC, N_PAD, D), jnp.float32),
    mesh=_mesh(),
    compiler_params=pltpu.CompilerParams(needs_layout_passes=False),
    scratch_types=[
        pltpu.VMEM_SHARED((N_PAD, D), jnp.float32),
        pltpu.VMEM((CHUNK_R, B_E), jnp.int32),
        pltpu.VMEM((CHUNK_R, B_E), jnp.int32),
        pltpu.VMEM((ROWS_PER_TILE,), jnp.float32),
        *_SEMS,
    ],
  )


def _tc_a_body(x_ref, wl_ref, wr_ref, y_ref, r_ref):
  xb = x_ref[...]
  y_ref[...] = jnp.dot(xb, wl_ref[...], preferred_element_type=jnp.float32)
  r_ref[...] = jnp.dot(xb, wr_ref[...], preferred_element_type=jnp.float32)


def _tc_b_body(p_ref, r_ref, b_ref, wl_ref, wr_ref, y2_ref, r2_ref):
  sb = p_ref[0] + p_ref[1] + r_ref[...] + b_ref[...][None, :]
  h = jnp.where(sb > 0, sb, jnp.exp(sb) - 1.0)
  y2_ref[...] = jnp.dot(h, wl_ref[...], preferred_element_type=jnp.float32)
  r2_ref[...] = jnp.dot(h, wr_ref[...], preferred_element_type=jnp.float32)


def _tc_c_body(p_ref, r_ref, b_ref, o_ref):
  sb = p_ref[0] + p_ref[1] + r_ref[...] + b_ref[...][None, :]
  o_ref[...] = jnp.where(sb > 0, sb, jnp.exp(sb) - 1.0)


_row_spec = pl.BlockSpec((ROW_BLK, D), lambda i: (i, 0))
_p_spec = pl.BlockSpec((NC, ROW_BLK, D), lambda i: (0, i, 0))
_w_spec = pl.BlockSpec((D, D), lambda i: (0, 0))
_b_spec = pl.BlockSpec((D,), lambda i: (0,))

_tc_a = pl.pallas_call(
    _tc_a_body,
    grid=(N // ROW_BLK,),
    in_specs=[_row_spec, _w_spec, _w_spec],
    out_specs=[_row_spec, _row_spec],
    out_shape=[jax.ShapeDtypeStruct((N, D), jnp.float32)] * 2,
)

_tc_b = pl.pallas_call(
    _tc_b_body,
    grid=(N // ROW_BLK,),
    in_specs=[_p_spec, _row_spec, _b_spec, _w_spec, _w_spec],
    out_specs=[_row_spec, _row_spec],
    out_shape=[jax.ShapeDtypeStruct((N, D), jnp.float32)] * 2,
)

def _tc_prep_body(ei_ref, src2_ref, dst2_ref):
  src = ei_ref[0].reshape(E // B_E, B_E)
  dst = ei_ref[1].reshape(E // B_E, B_E)
  pad = jax.lax.broadcasted_iota(jnp.int32, (NPAD_ROWS, B_E), 0) * B_E + \
      jax.lax.broadcasted_iota(jnp.int32, (NPAD_ROWS, B_E), 1)
  src2_ref[...] = jnp.concatenate([src, pad % N], axis=0)
  dst2_ref[...] = jnp.concatenate([dst, N + pad % (N_PAD - N)], axis=0)


_tc_prep = pl.pallas_call(
    _tc_prep_body,
    in_specs=[pl.BlockSpec((2, E), lambda: (0, 0))],
    out_specs=[pl.BlockSpec((E_PAD // B_E, B_E), lambda: (0, 0))] * 2,
    out_shape=[jax.ShapeDtypeStruct((E_PAD // B_E, B_E), jnp.int32)] * 2,
)

_tc_c = pl.pallas_call(
    _tc_c_body,
    grid=(N // ROW_BLK,),
    in_specs=[_p_spec, _row_spec, _b_spec],
    out_specs=_row_spec,
    out_shape=jax.ShapeDtypeStruct((N, D), jnp.float32),
)


def kernel(x, edge_index, Wl1, bl1, Wr1, Wl2, bl2, Wr2):
  src2, dst2 = _tc_prep(edge_index.astype(jnp.int32))

  inv = _sc_cnt()(dst2)
  y1, r1 = _tc_a(x, Wl1, Wr1)
  p1 = _sc_agg2()(y1, src2, dst2, inv)
  y2, r2 = _tc_b(p1, r1, bl1, Wl2, Wr2)
  p2 = _sc_agg2()(y2, src2, dst2, inv)
  return _tc_c(p2, r2, bl2)


# speedup vs baseline: 3.8230x; 1.0090x over previous
import functools

import jax
import jax.numpy as jnp
from jax import lax
from jax.experimental import pallas as pl
from jax.experimental.pallas import tpu as pltpu
from jax.experimental.pallas import tpu_sc as plsc

N = 10000
E = 320000
D = 128
L = 16
NC = 2
NS = 16
N_PAD = 10240
ROWS_PER_TILE = N_PAD // NS
E_PAD = 327680
E_TILE = E_PAD // (NC * NS)
B_E = 128
N_EBATCH = E_TILE // B_E
NBUF = 2
CHUNK_R = 16
NCHUNK = N_EBATCH // CHUNK_R
E_CNT_TILE = E_PAD // NS
CNT_ROWS = E_CNT_TILE // B_E
CNT_CHUNK_ROWS = 16
CROWS = N_PAD // B_E
CROWS_PER_TILE = CROWS // NS
ZROWS = 64
ROW_BLK = 1000
NPAD_ROWS = (E_PAD - E) // B_E

_mesh = functools.partial(
    plsc.VectorSubcoreMesh,
    core_axis_name="c", subcore_axis_name="s", num_cores=NC, num_subcores=NS)


def _zero_rows(ref, nrows):
  zero16 = jnp.zeros((L,), jnp.float32)
  def row(i, _):
    def col(q, _):
      ref[i, pl.ds(q * L, L)] = zero16
      return 0
    return lax.fori_loop(0, D // L, col, 0)
  lax.fori_loop(0, nrows, row, 0)


def _edge_scatter_loop(y_hbm, src2_hbm, dst2_hbm, acc_sh, srcs, dst2_v,
                       rows_bufs, sems, ssems, wid):
  row0 = wid * N_EBATCH
  pltpu.sync_copy(dst2_hbm.at[pl.ds(row0, N_EBATCH)], dst2_v)
  pltpu.sync_copy(src2_hbm.at[pl.ds(row0, CHUNK_R)], srcs[0])
  for b in range(NBUF):
    pltpu.async_copy(y_hbm.at[srcs[0].at[b]], rows_bufs[b], sems[b])
  for ch in range(NCHUNK):
    cur, nxt = srcs[ch % 2], srcs[(ch + 1) % 2]
    base = ch * CHUNK_R
    if ch + 1 < NCHUNK:
      pltpu.async_copy(src2_hbm.at[pl.ds(row0 + base + CHUNK_R, CHUNK_R)],
                       nxt, ssems[(ch + 1) % 2])
    def mid(g0, _):
      for b in range(NBUF):
        gl = g0 * NBUF + b
        pltpu.make_async_copy(
            y_hbm.at[cur.at[gl]], rows_bufs[b], sems[b]).wait()
        pltpu.sync_copy(rows_bufs[b], acc_sh.at[dst2_v.at[base + gl]],
                        add=True)
        pltpu.async_copy(y_hbm.at[cur.at[gl + NBUF]], rows_bufs[b], sems[b])
      return 0
    lax.fori_loop(0, (CHUNK_R - NBUF) // NBUF, mid, 0)
    if ch + 1 < NCHUNK:
      pltpu.make_async_copy(src2_hbm.at[pl.ds(row0 + base + CHUNK_R, CHUNK_R)],
                            nxt, ssems[(ch + 1) % 2]).wait()
    for b in range(NBUF):
      gl = CHUNK_R - NBUF + b
      pltpu.make_async_copy(
          y_hbm.at[cur.at[gl]], rows_bufs[b], sems[b]).wait()
      pltpu.sync_copy(rows_bufs[b], acc_sh.at[dst2_v.at[base + gl]],
                      add=True)
      if ch + 1 < NCHUNK:
        pltpu.async_copy(y_hbm.at[nxt.at[b]], rows_bufs[b], sems[b])


def _scale_and_emit(acc_sh, inv_v, stages, p_hbm, c, r0, sems_i, sems_o):
  nch = ROWS_PER_TILE // ZROWS
  for b in range(2):
    pltpu.async_copy(acc_sh.at[pl.ds(r0 + b * ZROWS, ZROWS)], stages[b],
                     sems_i[b])
  def step(j0, _):
    for b in range(2):
      j = j0 * 2 + b
      pltpu.make_async_copy(acc_sh.at[pl.ds(r0 + j * ZROWS, ZROWS)],
                            stages[b], sems_i[b]).wait()
      def row(rr, _):
        idx16 = jnp.full((L,), j * ZROWS + rr, jnp.int32)
        g = plsc.load_gather(inv_v, [idx16])
        def col(q, _):
          stages[b][rr, pl.ds(q * L, L)] = stages[b][rr, pl.ds(q * L, L)] * g
          return 0
        return lax.fori_loop(0, D // L, col, 0)
      lax.fori_loop(0, ZROWS, row, 0)
      pltpu.async_copy(stages[b], p_hbm.at[c, pl.ds(r0 + j * ZROWS, ZROWS)],
                       sems_o[b])
      @pl.when(j + 2 < nch)
      def _():
        pltpu.make_async_copy(
            stages[b], p_hbm.at[c, pl.ds(r0 + j * ZROWS, ZROWS)],
            sems_o[b]).wait()
        pltpu.async_copy(acc_sh.at[pl.ds(r0 + (j + 2) * ZROWS, ZROWS)],
                         stages[b], sems_i[b])
    return 0
  lax.fori_loop(0, nch // 2, step, 0)
  for b in range(2):
    j = nch - 2 + b
    pltpu.make_async_copy(
        stages[b], p_hbm.at[c, pl.ds(r0 + j * ZROWS, ZROWS)],
        sems_o[b]).wait()


def _zero_acc(acc_sh, stage_v, r0, sem):
  nch = ROWS_PER_TILE // ZROWS
  def fire(j, _):
    pltpu.async_copy(stage_v, acc_sh.at[pl.ds(r0 + j * ZROWS, ZROWS)], sem)
    return 0
  lax.fori_loop(0, nch, fire, 0)
  def drain(j, _):
    pltpu.make_async_copy(stage_v, acc_sh.at[pl.ds(r0 + j * ZROWS, ZROWS)],
                          sem).wait()
    return 0
  lax.fori_loop(0, nch, drain, 0)


def _sc_cnt_body(dst2_hbm, inv_hbm, cnt_sh, cnt5_v, inv_v):
  c = lax.axis_index("c")
  s = lax.axis_index("s")
  r0 = s * ROWS_PER_TILE
  zero16 = jnp.zeros((L,), jnp.float32)
  ones16 = jnp.full((L,), 1.0, jnp.float32)

  def scoped(cntloc_v, dstbuf_v, iota_v):
    def zcnt(i, _):
      def zcntc(q, _):
        cntloc_v[i, pl.ds(q * L, L)] = zero16
        return 0
      return lax.fori_loop(0, B_E // L, zcntc, 0)
    lax.fori_loop(0, CROWS, zcnt, 0)
    pltpu.sync_copy(cntloc_v.at[pl.ds(0, CROWS_PER_TILE)],
                    cnt_sh.at[pl.ds(s * CROWS_PER_TILE, CROWS_PER_TILE)])

    t0r = s * CNT_ROWS
    def cnt_chunk(jc, _):
      pltpu.sync_copy(
          dst2_hbm.at[pl.ds(t0r + jc * CNT_CHUNK_ROWS, CNT_CHUNK_ROWS)],
          dstbuf_v)
      def cnt_row(rr, _):
        def cnt_col(q, _):
          d16 = dstbuf_v[rr, pl.ds(q * L, L)]
          row16 = lax.shift_right_logical(d16, 7)
          col16 = lax.bitwise_and(d16, B_E - 1)
          plsc.addupdate_scatter(cntloc_v, [row16, col16], ones16)
          return 0
        return lax.fori_loop(0, B_E // L, cnt_col, 0)
      return lax.fori_loop(0, CNT_CHUNK_ROWS, cnt_row, 0)
    lax.fori_loop(0, CNT_ROWS // CNT_CHUNK_ROWS, cnt_chunk, 0)

    def mkiota(k, _):
      iota_v[pl.ds(k * L, L)] = lax.iota(jnp.int32, L) + k * L
      return 0
    lax.fori_loop(0, CROWS // L, mkiota, 0)

    plsc.subcore_barrier()
    pltpu.sync_copy(cntloc_v, cnt_sh.at[iota_v], add=True)

  pl.run_scoped(
      scoped,
      pltpu.VMEM((CROWS, B_E), jnp.float32),
      pltpu.VMEM((CNT_CHUNK_ROWS, B_E), jnp.int32),
      pltpu.VMEM((CROWS,), jnp.int32),
  )

  plsc.subcore_barrier()

  pltpu.sync_copy(cnt_sh.at[pl.ds(s * CROWS_PER_TILE, CROWS_PER_TILE)],
                  cnt5_v)
  def invrow(j, _):
    def invcol(q, _):
      v = cnt5_v[j, pl.ds(q * L, L)]
      inv_v[pl.ds((j * (B_E // L) + q) * L, L)] = ones16 / jnp.maximum(v, ones16)
      return 0
    return lax.fori_loop(0, B_E // L, invcol, 0)
  lax.fori_loop(0, CROWS_PER_TILE, invrow, 0)

  @pl.when(c == 0)
  def _():
    pltpu.sync_copy(inv_v, inv_hbm.at[pl.ds(r0, ROWS_PER_TILE)])


@functools.lru_cache(maxsize=None)
def _sc_cnt():
  return pl.kernel(
    _sc_cnt_body,
    out_type=jax.ShapeDtypeStruct((N_PAD,), jnp.float32),
    mesh=_mesh(),
    compiler_params=pltpu.CompilerParams(needs_layout_passes=False),
    scratch_types=[
        pltpu.VMEM_SHARED((CROWS, B_E), jnp.float32),
        pltpu.VMEM((CROWS_PER_TILE, B_E), jnp.float32),
        pltpu.VMEM((ROWS_PER_TILE,), jnp.float32),
    ],
  )


def _sc_agg2_body(y_hbm, src2_hbm, dst2_hbm, inv_hbm, p_hbm,
                  acc_sh, srcA_v, srcB_v, dst2_v, inv_v,
                  sem0, sem1, sem2, sem3, sem4, sem5):
  c = lax.axis_index("c")
  s = lax.axis_index("s")
  r0 = s * ROWS_PER_TILE

  def phase_a(stage_v):
    _zero_rows(stage_v, ZROWS)
    _zero_acc(acc_sh, stage_v, r0, sem0)
  pl.run_scoped(phase_a, pltpu.VMEM((ZROWS, D), jnp.float32))
  pltpu.sync_copy(inv_hbm.at[pl.ds(r0, ROWS_PER_TILE)], inv_v)

  plsc.subcore_barrier()

  def phase_b(rows0, rows1):
    _edge_scatter_loop(y_hbm, src2_hbm, dst2_hbm, acc_sh, (srcA_v, srcB_v),
                       dst2_v, (rows0, rows1), (sem0, sem1), (sem4, sem5),
                       c * NS + s)
  pl.run_scoped(phase_b,
                pltpu.VMEM((B_E, D), jnp.float32),
                pltpu.VMEM((B_E, D), jnp.float32))

  plsc.subcore_barrier()

  def phase_c(st0, st1):
    _scale_and_emit(acc_sh, inv_v, (st0, st1), p_hbm, c, r0,
                    (sem0, sem1), (sem2, sem3))
  pl.run_scoped(phase_c, pltpu.VMEM((ZROWS, D), jnp.float32),
                pltpu.VMEM((ZROWS, D), jnp.float32))


_SEMS = [pltpu.SemaphoreType.DMA] * 6


@functools.lru_cache(maxsize=None)
def _sc_agg2():
  return pl.kernel(
    _sc_agg2_body,
    out_type=jax.ShapeDtypeStruct((NC, N_PAD, D), jnp.float32),
    mesh=_mesh(),
    compiler_params=pltpu.CompilerParams(needs_layout_passes=False),
    scratch_types=[
        pltpu.VMEM_SHARED((N_PAD, D), jnp.float32),
        pltpu.VMEM((CHUNK_R, B_E), jnp.int32),
        pltpu.VMEM((CHUNK_R, B_E), jnp.int32),
        pltpu.VMEM((N_EBATCH, B_E), jnp.int32),
        pltpu.VMEM((ROWS_PER_TILE,), jnp.float32),
        *_SEMS,
    ],
  )


def _tc_a_body(x_ref, wl_ref, wr_ref, y_ref, r_ref):
  xb = x_ref[...]
  y_ref[...] = jnp.dot(xb, wl_ref[...], preferred_element_type=jnp.float32)
  r_ref[...] = jnp.dot(xb, wr_ref[...], preferred_element_type=jnp.float32)


def _tc_b_body(p_ref, r_ref, b_ref, wl_ref, wr_ref, y2_ref, r2_ref):
  sb = p_ref[0] + p_ref[1] + r_ref[...] + b_ref[...][None, :]
  h = jnp.where(sb > 0, sb, jnp.exp(sb) - 1.0)
  y2_ref[...] = jnp.dot(h, wl_ref[...], preferred_element_type=jnp.float32)
  r2_ref[...] = jnp.dot(h, wr_ref[...], preferred_element_type=jnp.float32)


def _tc_c_body(p_ref, r_ref, b_ref, o_ref):
  sb = p_ref[0] + p_ref[1] + r_ref[...] + b_ref[...][None, :]
  o_ref[...] = jnp.where(sb > 0, sb, jnp.exp(sb) - 1.0)


_row_spec = pl.BlockSpec((ROW_BLK, D), lambda i: (i, 0))
_p_spec = pl.BlockSpec((NC, ROW_BLK, D), lambda i: (0, i, 0))
_w_spec = pl.BlockSpec((D, D), lambda i: (0, 0))
_b_spec = pl.BlockSpec((D,), lambda i: (0,))

_tc_a = pl.pallas_call(
    _tc_a_body,
    grid=(N // ROW_BLK,),
    in_specs=[_row_spec, _w_spec, _w_spec],
    out_specs=[_row_spec, _row_spec],
    out_shape=[jax.ShapeDtypeStruct((N, D), jnp.float32)] * 2,
)

_tc_b = pl.pallas_call(
    _tc_b_body,
    grid=(N // ROW_BLK,),
    in_specs=[_p_spec, _row_spec, _b_spec, _w_spec, _w_spec],
    out_specs=[_row_spec, _row_spec],
    out_shape=[jax.ShapeDtypeStruct((N, D), jnp.float32)] * 2,
)

def _tc_prep_body(ei_ref, src2_ref, dst2_ref):
  src = ei_ref[0].reshape(E // B_E, B_E)
  dst = ei_ref[1].reshape(E // B_E, B_E)
  pad = jax.lax.broadcasted_iota(jnp.int32, (NPAD_ROWS, B_E), 0) * B_E + \
      jax.lax.broadcasted_iota(jnp.int32, (NPAD_ROWS, B_E), 1)
  src2_ref[...] = jnp.concatenate([src, pad % N], axis=0)
  dst2_ref[...] = jnp.concatenate([dst, N + pad % (N_PAD - N)], axis=0)


_tc_prep = pl.pallas_call(
    _tc_prep_body,
    in_specs=[pl.BlockSpec((2, E), lambda: (0, 0))],
    out_specs=[pl.BlockSpec((E_PAD // B_E, B_E), lambda: (0, 0))] * 2,
    out_shape=[jax.ShapeDtypeStruct((E_PAD // B_E, B_E), jnp.int32)] * 2,
)

_tc_c = pl.pallas_call(
    _tc_c_body,
    grid=(N // ROW_BLK,),
    in_specs=[_p_spec, _row_spec, _b_spec],
    out_specs=_row_spec,
    out_shape=jax.ShapeDtypeStruct((N, D), jnp.float32),
)


def kernel(x, edge_index, Wl1, bl1, Wr1, Wl2, bl2, Wr2):
  src2, dst2 = _tc_prep(edge_index.astype(jnp.int32))

  inv = _sc_cnt()(dst2)
  y1, r1 = _tc_a(x, Wl1, Wr1)
  p1 = _sc_agg2()(y1, src2, dst2, inv)
  y2, r2 = _tc_b(p1, r1, bl1, Wl2, Wr2)
  p2 = _sc_agg2()(y2, src2, dst2, inv)
  return _tc_c(p2, r2, bl2)


# speedup vs baseline: 3.8366x; 1.0036x over previous
import functools

import jax
import jax.numpy as jnp
from jax import lax
from jax.experimental import pallas as pl
from jax.experimental.pallas import tpu as pltpu
from jax.experimental.pallas import tpu_sc as plsc

N = 10000
E = 320000
D = 128
L = 16
NC = 2
NS = 16
N_PAD = 10240
ROWS_PER_TILE = N_PAD // NS
E_PAD = 327680
E_TILE = E_PAD // (NC * NS)
B_E = 128
N_EBATCH = E_TILE // B_E
NBUF = 2
CHUNK_R = 16
NCHUNK = N_EBATCH // CHUNK_R
E_CNT_TILE = E_PAD // NS
CNT_ROWS = E_CNT_TILE // B_E
CNT_CHUNK_ROWS = 16
CROWS = N_PAD // B_E
CROWS_PER_TILE = CROWS // NS
ZROWS = 64
ROW_BLK = 1000
NPAD_ROWS = (E_PAD - E) // B_E

_mesh = functools.partial(
    plsc.VectorSubcoreMesh,
    core_axis_name="c", subcore_axis_name="s", num_cores=NC, num_subcores=NS)


def _zero_rows(ref, nrows):
  zero16 = jnp.zeros((L,), jnp.float32)
  def row(i, _):
    def col(q, _):
      ref[i, pl.ds(q * L, L)] = zero16
      return 0
    return lax.fori_loop(0, D // L, col, 0)
  lax.fori_loop(0, nrows, row, 0)


def _edge_scatter_loop(y_hbm, src2_hbm, dst2_hbm, acc_sh, srcs, dst2_v,
                       rows_bufs, sems, ssems, wid):
  row0 = wid * N_EBATCH
  pltpu.sync_copy(dst2_hbm.at[pl.ds(row0, N_EBATCH)], dst2_v)
  pltpu.sync_copy(src2_hbm.at[pl.ds(row0, CHUNK_R)], srcs[0])
  for b in range(NBUF):
    pltpu.async_copy(y_hbm.at[srcs[0].at[b]], rows_bufs[b], sems[b])
  plsc.subcore_barrier()
  for ch in range(NCHUNK):
    cur, nxt = srcs[ch % 2], srcs[(ch + 1) % 2]
    base = ch * CHUNK_R
    if ch + 1 < NCHUNK:
      pltpu.async_copy(src2_hbm.at[pl.ds(row0 + base + CHUNK_R, CHUNK_R)],
                       nxt, ssems[(ch + 1) % 2])
    def mid(g0, _):
      for b in range(NBUF):
        gl = g0 * NBUF + b
        pltpu.make_async_copy(
            y_hbm.at[cur.at[gl]], rows_bufs[b], sems[b]).wait()
        pltpu.sync_copy(rows_bufs[b], acc_sh.at[dst2_v.at[base + gl]],
                        add=True)
        pltpu.async_copy(y_hbm.at[cur.at[gl + NBUF]], rows_bufs[b], sems[b])
      return 0
    lax.fori_loop(0, (CHUNK_R - NBUF) // NBUF, mid, 0)
    if ch + 1 < NCHUNK:
      pltpu.make_async_copy(src2_hbm.at[pl.ds(row0 + base + CHUNK_R, CHUNK_R)],
                            nxt, ssems[(ch + 1) % 2]).wait()
    for b in range(NBUF):
      gl = CHUNK_R - NBUF + b
      pltpu.make_async_copy(
          y_hbm.at[cur.at[gl]], rows_bufs[b], sems[b]).wait()
      pltpu.sync_copy(rows_bufs[b], acc_sh.at[dst2_v.at[base + gl]],
                      add=True)
      if ch + 1 < NCHUNK:
        pltpu.async_copy(y_hbm.at[nxt.at[b]], rows_bufs[b], sems[b])


def _scale_and_emit(acc_sh, inv_v, stages, p_hbm, c, r0, sems_i, sems_o):
  nch = ROWS_PER_TILE // ZROWS
  for b in range(2):
    pltpu.async_copy(acc_sh.at[pl.ds(r0 + b * ZROWS, ZROWS)], stages[b],
                     sems_i[b])
  def step(j0, _):
    for b in range(2):
      j = j0 * 2 + b
      pltpu.make_async_copy(acc_sh.at[pl.ds(r0 + j * ZROWS, ZROWS)],
                            stages[b], sems_i[b]).wait()
      def row(rr, _):
        idx16 = jnp.full((L,), j * ZROWS + rr, jnp.int32)
        g = plsc.load_gather(inv_v, [idx16])
        def col(q, _):
          stages[b][rr, pl.ds(q * L, L)] = stages[b][rr, pl.ds(q * L, L)] * g
          return 0
        return lax.fori_loop(0, D // L, col, 0)
      lax.fori_loop(0, ZROWS, row, 0)
      pltpu.async_copy(stages[b], p_hbm.at[c, pl.ds(r0 + j * ZROWS, ZROWS)],
                       sems_o[b])
      @pl.when(j + 2 < nch)
      def _():
        pltpu.make_async_copy(
            stages[b], p_hbm.at[c, pl.ds(r0 + j * ZROWS, ZROWS)],
            sems_o[b]).wait()
        pltpu.async_copy(acc_sh.at[pl.ds(r0 + (j + 2) * ZROWS, ZROWS)],
                         stages[b], sems_i[b])
    return 0
  lax.fori_loop(0, nch // 2, step, 0)
  for b in range(2):
    j = nch - 2 + b
    pltpu.make_async_copy(
        stages[b], p_hbm.at[c, pl.ds(r0 + j * ZROWS, ZROWS)],
        sems_o[b]).wait()


def _zero_acc(acc_sh, stage_v, r0, sem):
  nch = ROWS_PER_TILE // ZROWS
  def fire(j, _):
    pltpu.async_copy(stage_v, acc_sh.at[pl.ds(r0 + j * ZROWS, ZROWS)], sem)
    return 0
  lax.fori_loop(0, nch, fire, 0)
  def drain(j, _):
    pltpu.make_async_copy(stage_v, acc_sh.at[pl.ds(r0 + j * ZROWS, ZROWS)],
                          sem).wait()
    return 0
  lax.fori_loop(0, nch, drain, 0)


def _sc_cnt_body(dst2_hbm, inv_hbm, cnt_sh, cnt5_v, inv_v):
  c = lax.axis_index("c")
  s = lax.axis_index("s")
  r0 = s * ROWS_PER_TILE
  zero16 = jnp.zeros((L,), jnp.float32)
  ones16 = jnp.full((L,), 1.0, jnp.float32)

  def scoped(cntloc_v, dstbuf_v, iota_v):
    def zcnt(i, _):
      def zcntc(q, _):
        cntloc_v[i, pl.ds(q * L, L)] = zero16
        return 0
      return lax.fori_loop(0, B_E // L, zcntc, 0)
    lax.fori_loop(0, CROWS, zcnt, 0)
    pltpu.sync_copy(cntloc_v.at[pl.ds(0, CROWS_PER_TILE)],
                    cnt_sh.at[pl.ds(s * CROWS_PER_TILE, CROWS_PER_TILE)])

    t0r = s * CNT_ROWS
    def cnt_chunk(jc, _):
      pltpu.sync_copy(
          dst2_hbm.at[pl.ds(t0r + jc * CNT_CHUNK_ROWS, CNT_CHUNK_ROWS)],
          dstbuf_v)
      def cnt_row(rr, _):
        def cnt_col(q, _):
          d16 = dstbuf_v[rr, pl.ds(q * L, L)]
          row16 = lax.shift_right_logical(d16, 7)
          col16 = lax.bitwise_and(d16, B_E - 1)
          plsc.addupdate_scatter(cntloc_v, [row16, col16], ones16)
          return 0
        return lax.fori_loop(0, B_E // L, cnt_col, 0)
      return lax.fori_loop(0, CNT_CHUNK_ROWS, cnt_row, 0)
    lax.fori_loop(0, CNT_ROWS // CNT_CHUNK_ROWS, cnt_chunk, 0)

    def mkiota(k, _):
      iota_v[pl.ds(k * L, L)] = lax.iota(jnp.int32, L) + k * L
      return 0
    lax.fori_loop(0, CROWS // L, mkiota, 0)

    plsc.subcore_barrier()
    pltpu.sync_copy(cntloc_v, cnt_sh.at[iota_v], add=True)

  pl.run_scoped(
      scoped,
      pltpu.VMEM((CROWS, B_E), jnp.float32),
      pltpu.VMEM((CNT_CHUNK_ROWS, B_E), jnp.int32),
      pltpu.VMEM((CROWS,), jnp.int32),
  )

  plsc.subcore_barrier()

  pltpu.sync_copy(cnt_sh.at[pl.ds(s * CROWS_PER_TILE, CROWS_PER_TILE)],
                  cnt5_v)
  def invrow(j, _):
    def invcol(q, _):
      v = cnt5_v[j, pl.ds(q * L, L)]
      inv_v[pl.ds((j * (B_E // L) + q) * L, L)] = ones16 / jnp.maximum(v, ones16)
      return 0
    return lax.fori_loop(0, B_E // L, invcol, 0)
  lax.fori_loop(0, CROWS_PER_TILE, invrow, 0)

  @pl.when(c == 0)
  def _():
    pltpu.sync_copy(inv_v, inv_hbm.at[pl.ds(r0, ROWS_PER_TILE)])


@functools.lru_cache(maxsize=None)
def _sc_cnt():
  return pl.kernel(
    _sc_cnt_body,
    out_type=jax.ShapeDtypeStruct((N_PAD,), jnp.float32),
    mesh=_mesh(),
    compiler_params=pltpu.CompilerParams(needs_layout_passes=False),
    scratch_types=[
        pltpu.VMEM_SHARED((CROWS, B_E), jnp.float32),
        pltpu.VMEM((CROWS_PER_TILE, B_E), jnp.float32),
        pltpu.VMEM((ROWS_PER_TILE,), jnp.float32),
    ],
  )


def _sc_agg2_body(y_hbm, src2_hbm, dst2_hbm, inv_hbm, p_hbm,
                  acc_sh, srcA_v, srcB_v, dst2_v, inv_v,
                  sem0, sem1, sem2, sem3, sem4, sem5):
  c = lax.axis_index("c")
  s = lax.axis_index("s")
  r0 = s * ROWS_PER_TILE

  def phase_a(stage_v):
    _zero_rows(stage_v, ZROWS)
    _zero_acc(acc_sh, stage_v, r0, sem0)
  pl.run_scoped(phase_a, pltpu.VMEM((ZROWS, D), jnp.float32))
  pltpu.sync_copy(inv_hbm.at[pl.ds(r0, ROWS_PER_TILE)], inv_v)

  def phase_b(rows0, rows1):
    _edge_scatter_loop(y_hbm, src2_hbm, dst2_hbm, acc_sh, (srcA_v, srcB_v),
                       dst2_v, (rows0, rows1), (sem0, sem1), (sem4, sem5),
                       c * NS + s)
  pl.run_scoped(phase_b,
                pltpu.VMEM((B_E, D), jnp.float32),
                pltpu.VMEM((B_E, D), jnp.float32))

  plsc.subcore_barrier()

  def phase_c(st0, st1):
    _scale_and_emit(acc_sh, inv_v, (st0, st1), p_hbm, c, r0,
                    (sem0, sem1), (sem2, sem3))
  pl.run_scoped(phase_c, pltpu.VMEM((ZROWS, D), jnp.float32),
                pltpu.VMEM((ZROWS, D), jnp.float32))


_SEMS = [pltpu.SemaphoreType.DMA] * 6


@functools.lru_cache(maxsize=None)
def _sc_agg2():
  return pl.kernel(
    _sc_agg2_body,
    out_type=jax.ShapeDtypeStruct((NC, N_PAD, D), jnp.float32),
    mesh=_mesh(),
    compiler_params=pltpu.CompilerParams(needs_layout_passes=False),
    scratch_types=[
        pltpu.VMEM_SHARED((N_PAD, D), jnp.float32),
        pltpu.VMEM((CHUNK_R, B_E), jnp.int32),
        pltpu.VMEM((CHUNK_R, B_E), jnp.int32),
        pltpu.VMEM((N_EBATCH, B_E), jnp.int32),
        pltpu.VMEM((ROWS_PER_TILE,), jnp.float32),
        *_SEMS,
    ],
  )


def _tc_a_body(x_ref, wl_ref, wr_ref, y_ref, r_ref):
  xb = x_ref[...]
  y_ref[...] = jnp.dot(xb, wl_ref[...], preferred_element_type=jnp.float32)
  r_ref[...] = jnp.dot(xb, wr_ref[...], preferred_element_type=jnp.float32)


def _tc_b_body(p_ref, r_ref, b_ref, wl_ref, wr_ref, y2_ref, r2_ref):
  sb = p_ref[0] + p_ref[1] + r_ref[...] + b_ref[...][None, :]
  h = jnp.where(sb > 0, sb, jnp.exp(sb) - 1.0)
  y2_ref[...] = jnp.dot(h, wl_ref[...], preferred_element_type=jnp.float32)
  r2_ref[...] = jnp.dot(h, wr_ref[...], preferred_element_type=jnp.float32)


def _tc_c_body(p_ref, r_ref, b_ref, o_ref):
  sb = p_ref[0] + p_ref[1] + r_ref[...] + b_ref[...][None, :]
  o_ref[...] = jnp.where(sb > 0, sb, jnp.exp(sb) - 1.0)


_row_spec = pl.BlockSpec((ROW_BLK, D), lambda i: (i, 0))
_p_spec = pl.BlockSpec((NC, ROW_BLK, D), lambda i: (0, i, 0))
_w_spec = pl.BlockSpec((D, D), lambda i: (0, 0))
_b_spec = pl.BlockSpec((D,), lambda i: (0,))

_tc_a = pl.pallas_call(
    _tc_a_body,
    grid=(N // ROW_BLK,),
    in_specs=[_row_spec, _w_spec, _w_spec],
    out_specs=[_row_spec, _row_spec],
    out_shape=[jax.ShapeDtypeStruct((N, D), jnp.float32)] * 2,
)

_tc_b = pl.pallas_call(
    _tc_b_body,
    grid=(N // ROW_BLK,),
    in_specs=[_p_spec, _row_spec, _b_spec, _w_spec, _w_spec],
    out_specs=[_row_spec, _row_spec],
    out_shape=[jax.ShapeDtypeStruct((N, D), jnp.float32)] * 2,
)

def _tc_prep_body(ei_ref, src2_ref, dst2_ref):
  src = ei_ref[0].reshape(E // B_E, B_E)
  dst = ei_ref[1].reshape(E // B_E, B_E)
  pad = jax.lax.broadcasted_iota(jnp.int32, (NPAD_ROWS, B_E), 0) * B_E + \
      jax.lax.broadcasted_iota(jnp.int32, (NPAD_ROWS, B_E), 1)
  src2_ref[...] = jnp.concatenate([src, pad % N], axis=0)
  dst2_ref[...] = jnp.concatenate([dst, N + pad % (N_PAD - N)], axis=0)


_tc_prep = pl.pallas_call(
    _tc_prep_body,
    in_specs=[pl.BlockSpec((2, E), lambda: (0, 0))],
    out_specs=[pl.BlockSpec((E_PAD // B_E, B_E), lambda: (0, 0))] * 2,
    out_shape=[jax.ShapeDtypeStruct((E_PAD // B_E, B_E), jnp.int32)] * 2,
)

_tc_c = pl.pallas_call(
    _tc_c_body,
    grid=(N // ROW_BLK,),
    in_specs=[_p_spec, _row_spec, _b_spec],
    out_specs=_row_spec,
    out_shape=jax.ShapeDtypeStruct((N, D), jnp.float32),
)


def kernel(x, edge_index, Wl1, bl1, Wr1, Wl2, bl2, Wr2):
  src2, dst2 = _tc_prep(edge_index.astype(jnp.int32))

  inv = _sc_cnt()(dst2)
  y1, r1 = _tc_a(x, Wl1, Wr1)
  p1 = _sc_agg2()(y1, src2, dst2, inv)
  y2, r2 = _tc_b(p1, r1, bl1, Wl2, Wr2)
  p2 = _sc_agg2()(y2, src2, dst2, inv)
  return _tc_c(p2, r2, bl2)
